# Initial kernel scaffold; baseline (speedup 1.0000x reference)
import jax
import jax.numpy as jnp
from jax.experimental import pallas as pl


def kernel(x, edge_index, W1, b1, W2, b2, W3, b3, g1, be1, g2, be2, g3, be3, fcW1, fcb1, fcW2, fcb2):
    raise NotImplementedError("write your pallas kernel here")


# speedup vs baseline: 19.6983x; 19.6983x over previous
import functools

import jax
import jax.numpy as jnp
from jax import lax
from jax.experimental import pallas as pl
from jax.experimental.pallas import tpu as pltpu
from jax.experimental.pallas import tpu_sc as plsc

_NC = 2
_NS = 16
_K = 80
_R = 1000
_EPS = 1e-5


def _leaky(v):
    return jnp.where(v >= 0, v, 0.01 * v)


@functools.lru_cache(maxsize=None)
def _deg_kernel(N, E):
    NW = _NC * _NS
    epw = E // NW
    nb = epw // _K
    assert E % (NW * _K) == 0 and nb % 2 == 1 and nb >= 3
    npairs = nb // 2
    mesh = plsc.VectorSubcoreMesh(core_axis_name="c", subcore_axis_name="s")

    @functools.partial(
        pl.kernel,
        mesh=mesh,
        out_type=jax.ShapeDtypeStruct((_NC * N,), jnp.float32),
        scratch_types=[
            pltpu.VMEM((_K,), jnp.int32),
            pltpu.VMEM((_K,), jnp.int32),
            pltpu.VMEM((_K,), jnp.float32),
            pltpu.VMEM((N,), jnp.float32),
            pltpu.VMEM_SHARED((N,), jnp.float32),
            pltpu.SemaphoreType.DMA,
            pltpu.SemaphoreType.DMA,
        ],
    )
    def deg(dst, zeros, out, idx0, idx1, ones, stage, acc, sem0, sem1):
        c = lax.axis_index("c")
        s = lax.axis_index("s")
        base = (c * _NS + s) * epw

        @pl.when(s == 0)
        def _():
            pltpu.sync_copy(zeros, stage)
            pltpu.sync_copy(stage, acc)

        for j in range(_K // 16):
            ones[pl.ds(16 * j, 16)] = jnp.ones((16,), jnp.float32)
        plsc.subcore_barrier()

        def load(i, ref, sem):
            off = pl.multiple_of(base + i * _K, 8)
            pltpu.async_copy(dst.at[pl.ds(off, _K)], ref, sem)

        def scat(ref, sem):
            pltpu.make_async_copy(dst.at[pl.ds(0, _K)], ref, sem).wait()
            pltpu.sync_copy(ones, acc.at[ref], add=True)

        load(0, idx0, sem0)

        def pair(pi, carry):
            i = 2 * pi
            load(i + 1, idx1, sem1)
            scat(idx0, sem0)
            load(i + 2, idx0, sem0)
            scat(idx1, sem1)
            return carry

        lax.fori_loop(0, npairs, pair, 0)
        scat(idx0, sem0)

        plsc.subcore_barrier()

        @pl.when(s == 0)
        def _():
            coff = pl.multiple_of(c * N, 8)
            pltpu.sync_copy(acc, stage)
            pltpu.sync_copy(stage, out.at[pl.ds(coff, N)])

    return deg


@functools.lru_cache(maxsize=None)
def _scatter_kernel(N, F, E):
    NW = _NC * _NS
    epw = E // NW
    nb = epw // _K
    assert E % (NW * _K) == 0 and nb % 2 == 1 and nb >= 3
    rpt = (-(-N // _NS) + 7) // 8 * 8
    rpt_last = N - (_NS - 1) * rpt
    assert rpt_last > 0
    npairs = (nb - 3) // 2
    mesh = plsc.VectorSubcoreMesh(core_axis_name="c", subcore_axis_name="s")

    @functools.partial(
        pl.kernel,
        mesh=mesh,
        out_type=jax.ShapeDtypeStruct((_NC, N, F), jnp.float32),
        scratch_types=[
            pltpu.VMEM((_K,), jnp.int32),
            pltpu.VMEM((_K,), jnp.int32),
            pltpu.VMEM((_K,), jnp.int32),
            pltpu.VMEM((_K,), jnp.int32),
            pltpu.VMEM((_K, F), jnp.float32),
            pltpu.VMEM((_K, F), jnp.float32),
            pltpu.VMEM_SHARED((N, F), jnp.float32),
            pltpu.SemaphoreType.DMA,
            pltpu.SemaphoreType.DMA,
            pltpu.SemaphoreType.DMA,
            pltpu.SemaphoreType.DMA,
            pltpu.SemaphoreType.DMA,
            pltpu.SemaphoreType.DMA,
        ],
    )
    def scatter(hp, src, dst, out, si0, si1, di0, di1, rows0, rows1, acc,
                ss0, ss1, sd0, sd1, sg0, sg1):
        c = lax.axis_index("c")
        s = lax.axis_index("s")
        base = (c * _NS + s) * epw

        @pl.when(s < _NS - 1)
        def _():
            off = pl.multiple_of(s * rpt, 8)
            pltpu.sync_copy(hp.at[pl.ds(off, rpt)], acc.at[pl.ds(off, rpt)])

        @pl.when(s == _NS - 1)
        def _():
            off = (_NS - 1) * rpt
            pltpu.sync_copy(hp.at[pl.ds(off, rpt_last)],
                            acc.at[pl.ds(off, rpt_last)])

        plsc.subcore_barrier()

        def srcload(i, ref, sem):
            off = pl.multiple_of(base + i * _K, 8)
            pltpu.async_copy(src.at[pl.ds(off, _K)], ref, sem)

        def dstload(i, ref, sem):
            off = pl.multiple_of(base + i * _K, 8)
            pltpu.async_copy(dst.at[pl.ds(off, _K)], ref, sem)

        def iwait(ref, sem):
            pltpu.make_async_copy(src.at[pl.ds(0, _K)], ref, sem).wait()

        def gwait(rref, sem):
            pltpu.make_async_copy(hp.at[pl.ds(0, _K)], rref, sem).wait()

        slot0 = (si0, di0, rows0, ss0, sd0, sg0)
        slot1 = (si1, di1, rows1, ss1, sd1, sg1)

        def half(i, A, B, load_next):
            siA, diA, rA, ssA, sdA, sgA = A
            siB, diB, rB, ssB, sdB, sgB = B
            gwait(rA, sgA)
            iwait(siB, ssB)
            pltpu.async_copy(hp.at[siB], rB, sgB)
            if load_next:
                srcload(i + 2, siA, ssA)
            iwait(diA, sdA)
            pltpu.sync_copy(rA, acc.at[diA], add=True)
            if load_next:
                dstload(i + 2, diA, sdA)

        srcload(0, si0, ss0)
        dstload(0, di0, sd0)
        iwait(si0, ss0)
        pltpu.async_copy(hp.at[si0], rows0, sg0)
        srcload(1, si1, ss1)
        dstload(1, di1, sd1)

        def pair(pi, carry):
            i = 2 * pi
            half(i, slot0, slot1, True)
            half(i + 1, slot1, slot0, True)
            return carry

        lax.fori_loop(0, npairs, pair, 0)
        half(nb - 3, slot0, slot1, True)
        half(nb - 2, slot1, slot0, False)
        gwait(rows0, sg0)
        iwait(di0, sd0)
        pltpu.sync_copy(rows0, acc.at[di0], add=True)

        plsc.subcore_barrier()

        @pl.when(s < _NS - 1)
        def _():
            off = pl.multiple_of(s * rpt, 8)
            pltpu.sync_copy(acc.at[pl.ds(off, rpt)],
                            out.at[c, pl.ds(off, rpt)])

        @pl.when(s == _NS - 1)
        def _():
            off = (_NS - 1) * rpt
            pltpu.sync_copy(acc.at[pl.ds(off, rpt_last)],
                            out.at[c, pl.ds(off, rpt_last)])

    return scatter


def _input_proj(x, W1, degp):
    N, F = x.shape
    nbk = N // _R

    def body(x_ref, w_ref, d_ref, hp_ref, dinv_ref):
        deg = d_ref[0] + d_ref[1] + 1.0
        dv = lax.rsqrt(deg)
        dinv_ref[...] = dv
        hp_ref[...] = dv * jnp.dot(x_ref[...], w_ref[...],
                                   preferred_element_type=jnp.float32)

    return pl.pallas_call(
        body,
        grid=(nbk,),
        in_specs=[
            pl.BlockSpec((_R, F), lambda i: (i, 0)),
            pl.BlockSpec((F, F), lambda i: (0, 0)),
            pl.BlockSpec((2, _R, 1), lambda i: (0, i, 0)),
        ],
        out_specs=[
            pl.BlockSpec((_R, F), lambda i: (i, 0)),
            pl.BlockSpec((_R, 1), lambda i: (i, 0)),
        ],
        out_shape=[
            jax.ShapeDtypeStruct((N, F), jnp.float32),
            jax.ShapeDtypeStruct((N, 1), jnp.float32),
        ],
    )(x, W1, degp.reshape(_NC, N, 1))


def _layer_tail(sp, hp, dinv, b, g, be, res, Wn):
    N, F = hp.shape
    nbk = N // _R
    has_res = res is not None

    def body(*refs):
        if has_res:
            (sp_ref, hp_ref, dinv_ref, b_ref, g_ref, be_ref, res_ref, w_ref,
             z_ref, hpn_ref, y_s, ssum, ssq) = refs
        else:
            (sp_ref, hp_ref, dinv_ref, b_ref, g_ref, be_ref, w_ref,
             z_ref, hpn_ref, y_s, ssum, ssq) = refs
        p = pl.program_id(0)
        i = pl.program_id(1)

        @pl.when(p == 0)
        def _():
            y = (sp_ref[0] + sp_ref[1] - hp_ref[...]) * dinv_ref[...] + b_ref[...]
            y_s[pl.ds(i * _R, _R), :] = y

            @pl.when(i == 0)
            def _():
                ssum[...] = jnp.zeros_like(ssum)
                ssq[...] = jnp.zeros_like(ssq)

            ssum[...] += jnp.sum(y, axis=0, keepdims=True)
            ssq[...] += jnp.sum(y * y, axis=0, keepdims=True)

        @pl.when(p == 1)
        def _():
            m = ssum[...] / N
            v = ssq[...] / N - m * m
            y = y_s[pl.ds(i * _R, _R), :]
            yn = (y - m) * lax.rsqrt(v + _EPS) * g_ref[...] + be_ref[...]
            z = _leaky(yn)
            if has_res:
                z = z + res_ref[...]
            z_ref[...] = z
            hpn_ref[...] = dinv_ref[...] * jnp.dot(
                z, w_ref[...], preferred_element_type=jnp.float32)

    frozen = lambda p, i: (i * (1 - p) + (nbk - 1) * p, 0)
    sp_spec = pl.BlockSpec((_NC, _R, F), lambda p, i: (0,) + frozen(p, i))
    in_specs = [
        sp_spec,
        pl.BlockSpec((_R, F), frozen),
        pl.BlockSpec((_R, 1), lambda p, i: (i, 0)),
        pl.BlockSpec((1, F), lambda p, i: (0, 0)),
        pl.BlockSpec((1, F), lambda p, i: (0, 0)),
        pl.BlockSpec((1, F), lambda p, i: (0, 0)),
    ]
    args = [sp, hp, dinv, b.reshape(1, F), g.reshape(1, F), be.reshape(1, F)]
    if has_res:
        in_specs.append(pl.BlockSpec((_R, F), lambda p, i: (i * p, 0)))
        args.append(res)
    in_specs.append(pl.BlockSpec((F, F), lambda p, i: (0, 0)))
    args.append(Wn)

    return pl.pallas_call(
        body,
        grid=(2, nbk),
        in_specs=in_specs,
        out_specs=[
            pl.BlockSpec((_R, F), lambda p, i: (i * p, 0)),
            pl.BlockSpec((_R, F), lambda p, i: (i * p, 0)),
        ],
        out_shape=[
            jax.ShapeDtypeStruct((N, F), jnp.float32),
            jax.ShapeDtypeStruct((N, F), jnp.float32),
        ],
        scratch_shapes=[
            pltpu.VMEM((N, F), jnp.float32),
            pltpu.VMEM((1, F), jnp.float32),
            pltpu.VMEM((1, F), jnp.float32),
        ],
    )(*args)


def _head(sp, hp, dinv, b, g, be, res, fcW1, fcb1, fcW2, fcb2):
    N, F = hp.shape
    D = fcW2.shape[1]
    nbk = N // _R

    def body(sp_ref, hp_ref, dinv_ref, b_ref, g_ref, be_ref, res_ref,
             w1_ref, b1_ref, w2_ref, b2_ref, out_ref, y_s, ssum, ssq, zsum):
        p = pl.program_id(0)
        i = pl.program_id(1)

        @pl.when(p == 0)
        def _():
            y = (sp_ref[0] + sp_ref[1] - hp_ref[...]) * dinv_ref[...] + b_ref[...]
            y_s[pl.ds(i * _R, _R), :] = y

            @pl.when(i == 0)
            def _():
                ssum[...] = jnp.zeros_like(ssum)
                ssq[...] = jnp.zeros_like(ssq)

            ssum[...] += jnp.sum(y, axis=0, keepdims=True)
            ssq[...] += jnp.sum(y * y, axis=0, keepdims=True)

        @pl.when(p == 1)
        def _():
            m = ssum[...] / N
            v = ssq[...] / N - m * m
            y = y_s[pl.ds(i * _R, _R), :]
            yn = (y - m) * lax.rsqrt(v + _EPS) * g_ref[...] + be_ref[...]
            z = _leaky(yn) + res_ref[...]

            @pl.when(i == 0)
            def _():
                zsum[...] = jnp.zeros_like(zsum)

            zsum[...] += jnp.sum(z, axis=0, keepdims=True)

            @pl.when(i == nbk - 1)
            def _():
                pooled = zsum[...] / N
                h1 = _leaky(jnp.dot(pooled, w1_ref[...],
                                    preferred_element_type=jnp.float32)
                            + b1_ref[...])
                o = jnp.dot(h1, w2_ref[...],
                            preferred_element_type=jnp.float32) + b2_ref[...]
                nrm = jnp.sqrt(jnp.sum(o * o, axis=1, keepdims=True))
                out_ref[...] = o / jnp.maximum(nrm, 1e-12)

    frozen = lambda p, i: (i * (1 - p) + (nbk - 1) * p, 0)
    return pl.pallas_call(
        body,
        grid=(2, nbk),
        in_specs=[
            pl.BlockSpec((_NC, _R, F), lambda p, i: (0,) + frozen(p, i)),
            pl.BlockSpec((_R, F), frozen),
            pl.BlockSpec((_R, 1), lambda p, i: (i, 0)),
            pl.BlockSpec((1, F), lambda p, i: (0, 0)),
            pl.BlockSpec((1, F), lambda p, i: (0, 0)),
            pl.BlockSpec((1, F), lambda p, i: (0, 0)),
            pl.BlockSpec((_R, F), lambda p, i: (i * p, 0)),
            pl.BlockSpec((F, F), lambda p, i: (0, 0)),
            pl.BlockSpec((1, F), lambda p, i: (0, 0)),
            pl.BlockSpec((F, D), lambda p, i: (0, 0)),
            pl.BlockSpec((1, D), lambda p, i: (0, 0)),
        ],
        out_specs=pl.BlockSpec((1, D), lambda p, i: (0, 0)),
        out_shape=jax.ShapeDtypeStruct((1, D), jnp.float32),
        scratch_shapes=[
            pltpu.VMEM((N, F), jnp.float32),
            pltpu.VMEM((1, F), jnp.float32),
            pltpu.VMEM((1, F), jnp.float32),
            pltpu.VMEM((1, F), jnp.float32),
        ],
    )(sp, hp, dinv, b.reshape(1, F), g.reshape(1, F), be.reshape(1, F), res,
      fcW1, fcb1.reshape(1, F), fcW2, fcb2.reshape(1, D))


def kernel(x, edge_index, W1, b1, W2, b2, W3, b3, g1, be1, g2, be2, g3, be3,
           fcW1, fcb1, fcW2, fcb2):
    N, F = x.shape
    E = edge_index.shape[1]

    src = edge_index[0]
    dst = edge_index[1]
    degp = _deg_kernel(N, E)(dst, jnp.zeros((N,), jnp.float32)).reshape(_NC, N)
    hp1, dinv = _input_proj(x, W1, degp)

    scat = _scatter_kernel(N, F, E)
    sp1 = scat(hp1, src, dst)
    z1, hp2 = _layer_tail(sp1, hp1, dinv, b1, g1, be1, None, W2)
    sp2 = scat(hp2, src, dst)
    z2, hp3 = _layer_tail(sp2, hp2, dinv, b2, g2, be2, z1, W3)
    sp3 = scat(hp3, src, dst)
    return _head(sp3, hp3, dinv, b3, g3, be3, z2, fcW1, fcb1, fcW2, fcb2)


# speedup vs baseline: 22.9501x; 1.1651x over previous
import functools

import jax
import jax.numpy as jnp
from jax import lax
from jax.experimental import pallas as pl
from jax.experimental.pallas import tpu as pltpu
from jax.experimental.pallas import tpu_sc as plsc

_NC = 2
_NS = 16
_K = 128
_R = 1000
_EPS = 1e-5


def _leaky(v):
    return jnp.where(v >= 0, v, 0.01 * v)


@functools.lru_cache(maxsize=None)
def _deg_kernel(N, E):
    NW = _NC * _NS
    epw = E // NW
    nb = epw // _K
    rem = epw - nb * _K
    assert E % NW == 0 and nb % 2 == 0 and nb >= 4
    assert rem % 8 == 0
    mesh = plsc.VectorSubcoreMesh(core_axis_name="c", subcore_axis_name="s")

    @functools.partial(
        pl.kernel,
        mesh=mesh,
        out_type=jax.ShapeDtypeStruct((_NC * N,), jnp.float32),
        scratch_types=[
            pltpu.VMEM((_K,), jnp.int32),
            pltpu.VMEM((_K,), jnp.int32),
            pltpu.VMEM((max(rem, 8),), jnp.int32),
            pltpu.VMEM((_K,), jnp.float32),
            pltpu.VMEM((N,), jnp.float32),
            pltpu.VMEM_SHARED((N,), jnp.float32),
            pltpu.SemaphoreType.DMA,
            pltpu.SemaphoreType.DMA,
        ],
    )
    def deg(dst, zeros, out, idx0, idx1, idxt, ones, stage, acc, sem0, sem1):
        c = lax.axis_index("c")
        s = lax.axis_index("s")
        base = (c * _NS + s) * epw

        @pl.when(s == 0)
        def _():
            pltpu.sync_copy(zeros, stage)
            pltpu.sync_copy(stage, acc)

        for j in range(_K // 16):
            ones[pl.ds(16 * j, 16)] = jnp.ones((16,), jnp.float32)
        plsc.subcore_barrier()

        def load(i, ref, sem):
            off = pl.multiple_of(base + i * _K, 8)
            pltpu.async_copy(dst.at[pl.ds(off, _K)], ref, sem)

        def scat(ref, sem):
            pltpu.make_async_copy(dst.at[pl.ds(0, _K)], ref, sem).wait()
            pltpu.sync_copy(ones, acc.at[ref], add=True)

        load(0, idx0, sem0)

        def pair(pi, carry):
            i = 2 * pi
            load(i + 1, idx1, sem1)
            scat(idx0, sem0)
            load(i + 2, idx0, sem0)
            scat(idx1, sem1)
            return carry

        lax.fori_loop(0, nb // 2 - 1, pair, 0)
        load(nb - 1, idx1, sem1)
        scat(idx0, sem0)
        scat(idx1, sem1)
        if rem:
            roff = pl.multiple_of(base + nb * _K, 8)
            pltpu.sync_copy(dst.at[pl.ds(roff, rem)], idxt)
            pltpu.sync_copy(ones.at[pl.ds(0, rem)], acc.at[idxt], add=True)

        plsc.subcore_barrier()

        @pl.when(s == 0)
        def _():
            coff = pl.multiple_of(c * N, 8)
            pltpu.sync_copy(acc, stage)
            pltpu.sync_copy(stage, out.at[pl.ds(coff, N)])

    return deg


@functools.lru_cache(maxsize=None)
def _scatter_kernel(N, F, E):
    NW = _NC * _NS
    epw = E // NW
    nb = epw // _K
    rem = epw - nb * _K
    assert E % NW == 0 and nb % 2 == 0 and nb >= 6
    assert rem % 8 == 0
    rpt = (-(-N // _NS) + 7) // 8 * 8
    rpt_last = N - (_NS - 1) * rpt
    assert rpt_last > 0
    npairs = (nb - 2) // 2 - 1
    mesh = plsc.VectorSubcoreMesh(core_axis_name="c", subcore_axis_name="s")

    @functools.partial(
        pl.kernel,
        mesh=mesh,
        out_type=jax.ShapeDtypeStruct((_NC, N, F), jnp.float32),
        scratch_types=[
            pltpu.VMEM((_K,), jnp.int32),
            pltpu.VMEM((_K,), jnp.int32),
            pltpu.VMEM((_K,), jnp.int32),
            pltpu.VMEM((_K,), jnp.int32),
            pltpu.VMEM((max(rem, 8),), jnp.int32),
            pltpu.VMEM((max(rem, 8),), jnp.int32),
            pltpu.VMEM((_K, F), jnp.float32),
            pltpu.VMEM((_K, F), jnp.float32),
            pltpu.VMEM((max(rem, 8), F), jnp.float32),
            pltpu.VMEM_SHARED((N, F), jnp.float32),
            pltpu.SemaphoreType.DMA,
            pltpu.SemaphoreType.DMA,
            pltpu.SemaphoreType.DMA,
            pltpu.SemaphoreType.DMA,
            pltpu.SemaphoreType.DMA,
            pltpu.SemaphoreType.DMA,
            pltpu.SemaphoreType.DMA,
            pltpu.SemaphoreType.DMA,
        ],
    )
    def scatter(hp, src, dst, out, si0, si1, di0, di1, sit, dit,
                rows0, rows1, rowst, acc,
                ss0, ss1, sd0, sd1, sg0, sg1, sc0, sc1):
        c = lax.axis_index("c")
        s = lax.axis_index("s")
        base = (c * _NS + s) * epw

        @pl.when(s < _NS - 1)
        def _():
            off = pl.multiple_of(s * rpt, 8)
            pltpu.sync_copy(hp.at[pl.ds(off, rpt)], acc.at[pl.ds(off, rpt)])

        @pl.when(s == _NS - 1)
        def _():
            off = (_NS - 1) * rpt
            pltpu.sync_copy(hp.at[pl.ds(off, rpt_last)],
                            acc.at[pl.ds(off, rpt_last)])

        plsc.subcore_barrier()

        def srcload(i, ref, sem):
            off = pl.multiple_of(base + i * _K, 8)
            pltpu.async_copy(src.at[pl.ds(off, _K)], ref, sem)

        def dstload(i, ref, sem):
            off = pl.multiple_of(base + i * _K, 8)
            pltpu.async_copy(dst.at[pl.ds(off, _K)], ref, sem)

        def iwait(ref, sem):
            pltpu.make_async_copy(src.at[pl.ds(0, _K)], ref, sem).wait()

        def gwait(rref, sem):
            pltpu.make_async_copy(hp.at[pl.ds(0, _K)], rref, sem).wait()

        def scwait(rref, iref, sem):
            pltpu.make_async_copy(rref, acc.at[iref], sem).wait()

        slot0 = (si0, di0, rows0, ss0, sd0, sg0, sc0)
        slot1 = (si1, di1, rows1, ss1, sd1, sg1, sc1)

        def half(j, A, B, ld_src, ld_dst, do_scwait):
            siA, diA, rA, ssA, sdA, sgA, scA = A
            siB, diB, rB, ssB, sdB, sgB, scB = B
            gwait(rA, sgA)
            iwait(siB, ssB)
            if do_scwait:
                scwait(rB, diB, scB)
            pltpu.async_copy(hp.at[siB], rB, sgB)
            if ld_src:
                srcload(j + 2, siA, ssA)
            if ld_dst:
                dstload(j + 1, diB, sdB)
            iwait(diA, sdA)
            pltpu.async_copy(rA, acc.at[diA], scA, add=True)

        srcload(0, si0, ss0)
        dstload(0, di0, sd0)
        iwait(si0, ss0)
        pltpu.async_copy(hp.at[si0], rows0, sg0)
        srcload(1, si1, ss1)

        half(0, slot0, slot1, True, True, False)

        def pair(pi, carry):
            j = 2 * pi + 1
            half(j, slot1, slot0, True, True, True)
            half(j + 1, slot0, slot1, True, True, True)
            return carry

        lax.fori_loop(0, npairs, pair, 0)
        half(nb - 3, slot1, slot0, True, True, True)
        half(nb - 2, slot0, slot1, False, True, True)
        gwait(rows1, sg1)
        iwait(di1, sd1)
        pltpu.sync_copy(rows1, acc.at[di1], add=True)
        scwait(rows0, di0, sc0)

        if rem:
            roff = pl.multiple_of(base + nb * _K, 8)
            pltpu.sync_copy(src.at[pl.ds(roff, rem)], sit)
            pltpu.sync_copy(dst.at[pl.ds(roff, rem)], dit)
            pltpu.sync_copy(hp.at[sit], rowst)
            pltpu.sync_copy(rowst, acc.at[dit], add=True)

        plsc.subcore_barrier()

        @pl.when(s < _NS - 1)
        def _():
            off = pl.multiple_of(s * rpt, 8)
            pltpu.sync_copy(acc.at[pl.ds(off, rpt)],
                            out.at[c, pl.ds(off, rpt)])

        @pl.when(s == _NS - 1)
        def _():
            off = (_NS - 1) * rpt
            pltpu.sync_copy(acc.at[pl.ds(off, rpt_last)],
                            out.at[c, pl.ds(off, rpt_last)])

    return scatter


def _input_proj(x, W1, degp):
    N, F = x.shape
    nbk = N // _R

    def body(x_ref, w_ref, d_ref, hp_ref, dinv_ref):
        deg = d_ref[0] + d_ref[1] + 1.0
        dv = lax.rsqrt(deg)
        dinv_ref[...] = dv
        hp_ref[...] = dv * jnp.dot(x_ref[...], w_ref[...],
                                   preferred_element_type=jnp.float32)

    return pl.pallas_call(
        body,
        grid=(nbk,),
        in_specs=[
            pl.BlockSpec((_R, F), lambda i: (i, 0)),
            pl.BlockSpec((F, F), lambda i: (0, 0)),
            pl.BlockSpec((2, _R, 1), lambda i: (0, i, 0)),
        ],
        out_specs=[
            pl.BlockSpec((_R, F), lambda i: (i, 0)),
            pl.BlockSpec((_R, 1), lambda i: (i, 0)),
        ],
        out_shape=[
            jax.ShapeDtypeStruct((N, F), jnp.float32),
            jax.ShapeDtypeStruct((N, 1), jnp.float32),
        ],
    )(x, W1, degp.reshape(_NC, N, 1))


def _layer_tail(sp, hp, dinv, b, g, be, res, Wn):
    N, F = hp.shape
    nbk = N // _R
    has_res = res is not None

    def body(*refs):
        if has_res:
            (sp_ref, hp_ref, dinv_ref, b_ref, g_ref, be_ref, res_ref, w_ref,
             z_ref, hpn_ref, y_s, ssum, ssq) = refs
        else:
            (sp_ref, hp_ref, dinv_ref, b_ref, g_ref, be_ref, w_ref,
             z_ref, hpn_ref, y_s, ssum, ssq) = refs
        p = pl.program_id(0)
        i = pl.program_id(1)

        @pl.when(p == 0)
        def _():
            y = (sp_ref[0] + sp_ref[1] - hp_ref[...]) * dinv_ref[...] + b_ref[...]
            y_s[pl.ds(i * _R, _R), :] = y

            @pl.when(i == 0)
            def _():
                ssum[...] = jnp.zeros_like(ssum)
                ssq[...] = jnp.zeros_like(ssq)

            ssum[...] += jnp.sum(y, axis=0, keepdims=True)
            ssq[...] += jnp.sum(y * y, axis=0, keepdims=True)

        @pl.when(p == 1)
        def _():
            m = ssum[...] / N
            v = ssq[...] / N - m * m
            y = y_s[pl.ds(i * _R, _R), :]
            yn = (y - m) * lax.rsqrt(v + _EPS) * g_ref[...] + be_ref[...]
            z = _leaky(yn)
            if has_res:
                z = z + res_ref[...]
            z_ref[...] = z
            hpn_ref[...] = dinv_ref[...] * jnp.dot(
                z, w_ref[...], preferred_element_type=jnp.float32)

    frozen = lambda p, i: (i * (1 - p) + (nbk - 1) * p, 0)
    sp_spec = pl.BlockSpec((_NC, _R, F), lambda p, i: (0,) + frozen(p, i))
    in_specs = [
        sp_spec,
        pl.BlockSpec((_R, F), frozen),
        pl.BlockSpec((_R, 1), lambda p, i: (i, 0)),
        pl.BlockSpec((1, F), lambda p, i: (0, 0)),
        pl.BlockSpec((1, F), lambda p, i: (0, 0)),
        pl.BlockSpec((1, F), lambda p, i: (0, 0)),
    ]
    args = [sp, hp, dinv, b.reshape(1, F), g.reshape(1, F), be.reshape(1, F)]
    if has_res:
        in_specs.append(pl.BlockSpec((_R, F), lambda p, i: (i * p, 0)))
        args.append(res)
    in_specs.append(pl.BlockSpec((F, F), lambda p, i: (0, 0)))
    args.append(Wn)

    return pl.pallas_call(
        body,
        grid=(2, nbk),
        in_specs=in_specs,
        out_specs=[
            pl.BlockSpec((_R, F), lambda p, i: (i * p, 0)),
            pl.BlockSpec((_R, F), lambda p, i: (i * p, 0)),
        ],
        out_shape=[
            jax.ShapeDtypeStruct((N, F), jnp.float32),
            jax.ShapeDtypeStruct((N, F), jnp.float32),
        ],
        scratch_shapes=[
            pltpu.VMEM((N, F), jnp.float32),
            pltpu.VMEM((1, F), jnp.float32),
            pltpu.VMEM((1, F), jnp.float32),
        ],
    )(*args)


def _head(sp, hp, dinv, b, g, be, res, fcW1, fcb1, fcW2, fcb2):
    N, F = hp.shape
    D = fcW2.shape[1]
    nbk = N // _R

    def body(sp_ref, hp_ref, dinv_ref, b_ref, g_ref, be_ref, res_ref,
             w1_ref, b1_ref, w2_ref, b2_ref, out_ref, y_s, ssum, ssq, zsum):
        p = pl.program_id(0)
        i = pl.program_id(1)

        @pl.when(p == 0)
        def _():
            y = (sp_ref[0] + sp_ref[1] - hp_ref[...]) * dinv_ref[...] + b_ref[...]
            y_s[pl.ds(i * _R, _R), :] = y

            @pl.when(i == 0)
            def _():
                ssum[...] = jnp.zeros_like(ssum)
                ssq[...] = jnp.zeros_like(ssq)

            ssum[...] += jnp.sum(y, axis=0, keepdims=True)
            ssq[...] += jnp.sum(y * y, axis=0, keepdims=True)

        @pl.when(p == 1)
        def _():
            m = ssum[...] / N
            v = ssq[...] / N - m * m
            y = y_s[pl.ds(i * _R, _R), :]
            yn = (y - m) * lax.rsqrt(v + _EPS) * g_ref[...] + be_ref[...]
            z = _leaky(yn) + res_ref[...]

            @pl.when(i == 0)
            def _():
                zsum[...] = jnp.zeros_like(zsum)

            zsum[...] += jnp.sum(z, axis=0, keepdims=True)

            @pl.when(i == nbk - 1)
            def _():
                pooled = zsum[...] / N
                h1 = _leaky(jnp.dot(pooled, w1_ref[...],
                                    preferred_element_type=jnp.float32)
                            + b1_ref[...])
                o = jnp.dot(h1, w2_ref[...],
                            preferred_element_type=jnp.float32) + b2_ref[...]
                nrm = jnp.sqrt(jnp.sum(o * o, axis=1, keepdims=True))
                out_ref[...] = o / jnp.maximum(nrm, 1e-12)

    frozen = lambda p, i: (i * (1 - p) + (nbk - 1) * p, 0)
    return pl.pallas_call(
        body,
        grid=(2, nbk),
        in_specs=[
            pl.BlockSpec((_NC, _R, F), lambda p, i: (0,) + frozen(p, i)),
            pl.BlockSpec((_R, F), frozen),
            pl.BlockSpec((_R, 1), lambda p, i: (i, 0)),
            pl.BlockSpec((1, F), lambda p, i: (0, 0)),
            pl.BlockSpec((1, F), lambda p, i: (0, 0)),
            pl.BlockSpec((1, F), lambda p, i: (0, 0)),
            pl.BlockSpec((_R, F), lambda p, i: (i * p, 0)),
            pl.BlockSpec((F, F), lambda p, i: (0, 0)),
            pl.BlockSpec((1, F), lambda p, i: (0, 0)),
            pl.BlockSpec((F, D), lambda p, i: (0, 0)),
            pl.BlockSpec((1, D), lambda p, i: (0, 0)),
        ],
        out_specs=pl.BlockSpec((1, D), lambda p, i: (0, 0)),
        out_shape=jax.ShapeDtypeStruct((1, D), jnp.float32),
        scratch_shapes=[
            pltpu.VMEM((N, F), jnp.float32),
            pltpu.VMEM((1, F), jnp.float32),
            pltpu.VMEM((1, F), jnp.float32),
            pltpu.VMEM((1, F), jnp.float32),
        ],
    )(sp, hp, dinv, b.reshape(1, F), g.reshape(1, F), be.reshape(1, F), res,
      fcW1, fcb1.reshape(1, F), fcW2, fcb2.reshape(1, D))


def kernel(x, edge_index, W1, b1, W2, b2, W3, b3, g1, be1, g2, be2, g3, be3,
           fcW1, fcb1, fcW2, fcb2):
    N, F = x.shape
    E = edge_index.shape[1]

    src = edge_index[0]
    dst = edge_index[1]
    degp = _deg_kernel(N, E)(dst, jnp.zeros((N,), jnp.float32)).reshape(_NC, N)
    hp1, dinv = _input_proj(x, W1, degp)

    scat = _scatter_kernel(N, F, E)
    sp1 = scat(hp1, src, dst)
    z1, hp2 = _layer_tail(sp1, hp1, dinv, b1, g1, be1, None, W2)
    sp2 = scat(hp2, src, dst)
    z2, hp3 = _layer_tail(sp2, hp2, dinv, b2, g2, be2, z1, W3)
    sp3 = scat(hp3, src, dst)
    return _head(sp3, hp3, dinv, b3, g3, be3, z2, fcW1, fcb1, fcW2, fcb2)


# speedup vs baseline: 23.3779x; 1.0186x over previous
import functools

import jax
import jax.numpy as jnp
from jax import lax
from jax.experimental import pallas as pl
from jax.experimental.pallas import tpu as pltpu
from jax.experimental.pallas import tpu_sc as plsc

_NC = 2
_NS = 16
_K = 128
_R = 1000
_EPS = 1e-5


def _leaky(v):
    return jnp.where(v >= 0, v, 0.01 * v)


@functools.lru_cache(maxsize=None)
def _deg_kernel(N, E):
    NW = _NC * _NS
    epw = E // NW
    nb = epw // _K
    rem = epw - nb * _K
    assert E % NW == 0 and nb % 2 == 0 and nb >= 4
    assert rem % 8 == 0
    mesh = plsc.VectorSubcoreMesh(core_axis_name="c", subcore_axis_name="s")

    @functools.partial(
        pl.kernel,
        mesh=mesh,
        out_type=jax.ShapeDtypeStruct((_NC * N,), jnp.float32),
        scratch_types=[
            pltpu.VMEM((_K,), jnp.int32),
            pltpu.VMEM((_K,), jnp.int32),
            pltpu.VMEM((max(rem, 8),), jnp.int32),
            pltpu.VMEM((_K,), jnp.float32),
            pltpu.VMEM((N,), jnp.float32),
            pltpu.VMEM_SHARED((N,), jnp.float32),
            pltpu.SemaphoreType.DMA,
            pltpu.SemaphoreType.DMA,
        ],
    )
    def deg(ei, zeros, out, idx0, idx1, idxt, ones, stage, acc, sem0, sem1):
        c = lax.axis_index("c")
        s = lax.axis_index("s")
        base = E + (c * _NS + s) * epw

        @pl.when(s == 0)
        def _():
            pltpu.sync_copy(zeros, stage)
            pltpu.sync_copy(stage, acc)

        for j in range(_K // 16):
            ones[pl.ds(16 * j, 16)] = jnp.ones((16,), jnp.float32)
        plsc.subcore_barrier()

        def load(i, ref, sem):
            off = pl.multiple_of(base + i * _K, 8)
            pltpu.async_copy(ei.at[pl.ds(off, _K)], ref, sem)

        def scat(ref, sem):
            pltpu.make_async_copy(ei.at[pl.ds(0, _K)], ref, sem).wait()
            pltpu.sync_copy(ones, acc.at[ref], add=True)

        load(0, idx0, sem0)

        def pair(pi, carry):
            i = 2 * pi
            load(i + 1, idx1, sem1)
            scat(idx0, sem0)
            load(i + 2, idx0, sem0)
            scat(idx1, sem1)
            return carry

        lax.fori_loop(0, nb // 2 - 1, pair, 0)
        load(nb - 1, idx1, sem1)
        scat(idx0, sem0)
        scat(idx1, sem1)
        if rem:
            roff = pl.multiple_of(base + nb * _K, 8)
            pltpu.sync_copy(ei.at[pl.ds(roff, rem)], idxt)
            pltpu.sync_copy(ones.at[pl.ds(0, rem)], acc.at[idxt], add=True)

        plsc.subcore_barrier()

        @pl.when(s == 0)
        def _():
            coff = pl.multiple_of(c * N, 8)
            pltpu.sync_copy(acc, stage)
            pltpu.sync_copy(stage, out.at[pl.ds(coff, N)])

    return deg


@functools.lru_cache(maxsize=None)
def _scatter_kernel(N, F, E):
    NW = _NC * _NS
    epw = E // NW
    nb = epw // _K
    rem = epw - nb * _K
    assert E % NW == 0 and nb % 2 == 0 and nb >= 6
    assert rem % 8 == 0
    rpt = (-(-N // _NS) + 7) // 8 * 8
    rpt_last = N - (_NS - 1) * rpt
    assert rpt_last > 0
    npairs = (nb - 2) // 2 - 1
    mesh = plsc.VectorSubcoreMesh(core_axis_name="c", subcore_axis_name="s")

    @functools.partial(
        pl.kernel,
        mesh=mesh,
        out_type=jax.ShapeDtypeStruct((_NC, N, F), jnp.float32),
        scratch_types=[
            pltpu.VMEM((_K,), jnp.int32),
            pltpu.VMEM((_K,), jnp.int32),
            pltpu.VMEM((_K,), jnp.int32),
            pltpu.VMEM((_K,), jnp.int32),
            pltpu.VMEM((max(rem, 8),), jnp.int32),
            pltpu.VMEM((max(rem, 8),), jnp.int32),
            pltpu.VMEM((_K, F), jnp.float32),
            pltpu.VMEM((_K, F), jnp.float32),
            pltpu.VMEM((max(rem, 8), F), jnp.float32),
            pltpu.VMEM_SHARED((N, F), jnp.float32),
            pltpu.SemaphoreType.DMA,
            pltpu.SemaphoreType.DMA,
            pltpu.SemaphoreType.DMA,
            pltpu.SemaphoreType.DMA,
            pltpu.SemaphoreType.DMA,
            pltpu.SemaphoreType.DMA,
            pltpu.SemaphoreType.DMA,
            pltpu.SemaphoreType.DMA,
        ],
    )
    def scatter(hp, ei, out, si0, si1, di0, di1, sit, dit,
                rows0, rows1, rowst, acc,
                ss0, ss1, sd0, sd1, sg0, sg1, sc0, sc1):
        c = lax.axis_index("c")
        s = lax.axis_index("s")
        base = (c * _NS + s) * epw

        @pl.when(s < _NS - 1)
        def _():
            off = pl.multiple_of(s * rpt, 8)
            pltpu.sync_copy(hp.at[pl.ds(off, rpt)], acc.at[pl.ds(off, rpt)])

        @pl.when(s == _NS - 1)
        def _():
            off = (_NS - 1) * rpt
            pltpu.sync_copy(hp.at[pl.ds(off, rpt_last)],
                            acc.at[pl.ds(off, rpt_last)])

        plsc.subcore_barrier()

        def srcload(i, ref, sem):
            off = pl.multiple_of(base + i * _K, 8)
            pltpu.async_copy(ei.at[pl.ds(off, _K)], ref, sem)

        def dstload(i, ref, sem):
            off = pl.multiple_of(E + base + i * _K, 8)
            pltpu.async_copy(ei.at[pl.ds(off, _K)], ref, sem)

        def iwait(ref, sem):
            pltpu.make_async_copy(ei.at[pl.ds(0, _K)], ref, sem).wait()

        def gwait(rref, sem):
            pltpu.make_async_copy(hp.at[pl.ds(0, _K)], rref, sem).wait()

        def scwait(rref, iref, sem):
            pltpu.make_async_copy(rref, acc.at[iref], sem).wait()

        slot0 = (si0, di0, rows0, ss0, sd0, sg0, sc0)
        slot1 = (si1, di1, rows1, ss1, sd1, sg1, sc1)

        def half(j, A, B, ld_src, ld_dst, do_scwait):
            siA, diA, rA, ssA, sdA, sgA, scA = A
            siB, diB, rB, ssB, sdB, sgB, scB = B
            gwait(rA, sgA)
            iwait(siB, ssB)
            if do_scwait:
                scwait(rB, diB, scB)
            pltpu.async_copy(hp.at[siB], rB, sgB)
            if ld_src:
                srcload(j + 2, siA, ssA)
            if ld_dst:
                dstload(j + 1, diB, sdB)
            iwait(diA, sdA)
            pltpu.async_copy(rA, acc.at[diA], scA, add=True)

        srcload(0, si0, ss0)
        dstload(0, di0, sd0)
        iwait(si0, ss0)
        pltpu.async_copy(hp.at[si0], rows0, sg0)
        srcload(1, si1, ss1)

        half(0, slot0, slot1, True, True, False)

        def pair(pi, carry):
            j = 2 * pi + 1
            half(j, slot1, slot0, True, True, True)
            half(j + 1, slot0, slot1, True, True, True)
            return carry

        lax.fori_loop(0, npairs, pair, 0)
        half(nb - 3, slot1, slot0, True, True, True)
        half(nb - 2, slot0, slot1, False, True, True)
        gwait(rows1, sg1)
        iwait(di1, sd1)
        pltpu.sync_copy(rows1, acc.at[di1], add=True)
        scwait(rows0, di0, sc0)

        if rem:
            roff = pl.multiple_of(base + nb * _K, 8)
            pltpu.sync_copy(ei.at[pl.ds(roff, rem)], sit)
            pltpu.sync_copy(ei.at[pl.ds(E + roff, rem)], dit)
            pltpu.sync_copy(hp.at[sit], rowst)
            pltpu.sync_copy(rowst, acc.at[dit], add=True)

        plsc.subcore_barrier()

        @pl.when(s < _NS - 1)
        def _():
            off = pl.multiple_of(s * rpt, 8)
            pltpu.sync_copy(acc.at[pl.ds(off, rpt)],
                            out.at[c, pl.ds(off, rpt)])

        @pl.when(s == _NS - 1)
        def _():
            off = (_NS - 1) * rpt
            pltpu.sync_copy(acc.at[pl.ds(off, rpt_last)],
                            out.at[c, pl.ds(off, rpt_last)])

    return scatter


def _input_proj(x, W1, degp):
    N, F = x.shape
    nbk = N // _R

    def body(x_ref, w_ref, d_ref, hp_ref, dinv_ref):
        deg = d_ref[0] + d_ref[1] + 1.0
        dv = lax.rsqrt(deg)
        dinv_ref[...] = dv
        hp_ref[...] = dv * jnp.dot(x_ref[...], w_ref[...],
                                   preferred_element_type=jnp.float32)

    return pl.pallas_call(
        body,
        grid=(nbk,),
        in_specs=[
            pl.BlockSpec((_R, F), lambda i: (i, 0)),
            pl.BlockSpec((F, F), lambda i: (0, 0)),
            pl.BlockSpec((2, _R, 1), lambda i: (0, i, 0)),
        ],
        out_specs=[
            pl.BlockSpec((_R, F), lambda i: (i, 0)),
            pl.BlockSpec((_R, 1), lambda i: (i, 0)),
        ],
        out_shape=[
            jax.ShapeDtypeStruct((N, F), jnp.float32),
            jax.ShapeDtypeStruct((N, 1), jnp.float32),
        ],
    )(x, W1, degp.reshape(_NC, N, 1))


def _layer_tail(sp, hp, dinv, b, g, be, res, Wn):
    N, F = hp.shape
    nbk = N // _R
    has_res = res is not None

    def body(*refs):
        if has_res:
            (sp_ref, hp_ref, dinv_ref, b_ref, g_ref, be_ref, res_ref, w_ref,
             z_ref, hpn_ref, y_s, ssum, ssq) = refs
        else:
            (sp_ref, hp_ref, dinv_ref, b_ref, g_ref, be_ref, w_ref,
             z_ref, hpn_ref, y_s, ssum, ssq) = refs
        p = pl.program_id(0)
        i = pl.program_id(1)

        @pl.when(p == 0)
        def _():
            y = (sp_ref[0] + sp_ref[1] - hp_ref[...]) * dinv_ref[...] + b_ref[...]
            y_s[pl.ds(i * _R, _R), :] = y

            @pl.when(i == 0)
            def _():
                ssum[...] = jnp.zeros_like(ssum)
                ssq[...] = jnp.zeros_like(ssq)

            ssum[...] += jnp.sum(y, axis=0, keepdims=True)
            ssq[...] += jnp.sum(y * y, axis=0, keepdims=True)

        @pl.when(p == 1)
        def _():
            m = ssum[...] / N
            v = ssq[...] / N - m * m
            y = y_s[pl.ds(i * _R, _R), :]
            yn = (y - m) * lax.rsqrt(v + _EPS) * g_ref[...] + be_ref[...]
            z = _leaky(yn)
            if has_res:
                z = z + res_ref[...]
            z_ref[...] = z
            hpn_ref[...] = dinv_ref[...] * jnp.dot(
                z, w_ref[...], preferred_element_type=jnp.float32)

    frozen = lambda p, i: (i * (1 - p) + (nbk - 1) * p, 0)
    sp_spec = pl.BlockSpec((_NC, _R, F), lambda p, i: (0,) + frozen(p, i))
    in_specs = [
        sp_spec,
        pl.BlockSpec((_R, F), frozen),
        pl.BlockSpec((_R, 1), lambda p, i: (i, 0)),
        pl.BlockSpec((1, F), lambda p, i: (0, 0)),
        pl.BlockSpec((1, F), lambda p, i: (0, 0)),
        pl.BlockSpec((1, F), lambda p, i: (0, 0)),
    ]
    args = [sp, hp, dinv, b.reshape(1, F), g.reshape(1, F), be.reshape(1, F)]
    if has_res:
        in_specs.append(pl.BlockSpec((_R, F), lambda p, i: (i * p, 0)))
        args.append(res)
    in_specs.append(pl.BlockSpec((F, F), lambda p, i: (0, 0)))
    args.append(Wn)

    return pl.pallas_call(
        body,
        grid=(2, nbk),
        in_specs=in_specs,
        out_specs=[
            pl.BlockSpec((_R, F), lambda p, i: (i * p, 0)),
            pl.BlockSpec((_R, F), lambda p, i: (i * p, 0)),
        ],
        out_shape=[
            jax.ShapeDtypeStruct((N, F), jnp.float32),
            jax.ShapeDtypeStruct((N, F), jnp.float32),
        ],
        scratch_shapes=[
            pltpu.VMEM((N, F), jnp.float32),
            pltpu.VMEM((1, F), jnp.float32),
            pltpu.VMEM((1, F), jnp.float32),
        ],
    )(*args)


def _head(sp, hp, dinv, b, g, be, res, fcW1, fcb1, fcW2, fcb2):
    N, F = hp.shape
    D = fcW2.shape[1]
    nbk = N // _R

    def body(sp_ref, hp_ref, dinv_ref, b_ref, g_ref, be_ref, res_ref,
             w1_ref, b1_ref, w2_ref, b2_ref, out_ref, y_s, ssum, ssq, zsum):
        p = pl.program_id(0)
        i = pl.program_id(1)

        @pl.when(p == 0)
        def _():
            y = (sp_ref[0] + sp_ref[1] - hp_ref[...]) * dinv_ref[...] + b_ref[...]
            y_s[pl.ds(i * _R, _R), :] = y

            @pl.when(i == 0)
            def _():
                ssum[...] = jnp.zeros_like(ssum)
                ssq[...] = jnp.zeros_like(ssq)

            ssum[...] += jnp.sum(y, axis=0, keepdims=True)
            ssq[...] += jnp.sum(y * y, axis=0, keepdims=True)

        @pl.when(p == 1)
        def _():
            m = ssum[...] / N
            v = ssq[...] / N - m * m
            y = y_s[pl.ds(i * _R, _R), :]
            yn = (y - m) * lax.rsqrt(v + _EPS) * g_ref[...] + be_ref[...]
            z = _leaky(yn) + res_ref[...]

            @pl.when(i == 0)
            def _():
                zsum[...] = jnp.zeros_like(zsum)

            zsum[...] += jnp.sum(z, axis=0, keepdims=True)

            @pl.when(i == nbk - 1)
            def _():
                pooled = zsum[...] / N
                h1 = _leaky(jnp.dot(pooled, w1_ref[...],
                                    preferred_element_type=jnp.float32)
                            + b1_ref[...])
                o = jnp.dot(h1, w2_ref[...],
                            preferred_element_type=jnp.float32) + b2_ref[...]
                nrm = jnp.sqrt(jnp.sum(o * o, axis=1, keepdims=True))
                out_ref[...] = o / jnp.maximum(nrm, 1e-12)

    frozen = lambda p, i: (i * (1 - p) + (nbk - 1) * p, 0)
    return pl.pallas_call(
        body,
        grid=(2, nbk),
        in_specs=[
            pl.BlockSpec((_NC, _R, F), lambda p, i: (0,) + frozen(p, i)),
            pl.BlockSpec((_R, F), frozen),
            pl.BlockSpec((_R, 1), lambda p, i: (i, 0)),
            pl.BlockSpec((1, F), lambda p, i: (0, 0)),
            pl.BlockSpec((1, F), lambda p, i: (0, 0)),
            pl.BlockSpec((1, F), lambda p, i: (0, 0)),
            pl.BlockSpec((_R, F), lambda p, i: (i * p, 0)),
            pl.BlockSpec((F, F), lambda p, i: (0, 0)),
            pl.BlockSpec((1, F), lambda p, i: (0, 0)),
            pl.BlockSpec((F, D), lambda p, i: (0, 0)),
            pl.BlockSpec((1, D), lambda p, i: (0, 0)),
        ],
        out_specs=pl.BlockSpec((1, D), lambda p, i: (0, 0)),
        out_shape=jax.ShapeDtypeStruct((1, D), jnp.float32),
        scratch_shapes=[
            pltpu.VMEM((N, F), jnp.float32),
            pltpu.VMEM((1, F), jnp.float32),
            pltpu.VMEM((1, F), jnp.float32),
            pltpu.VMEM((1, F), jnp.float32),
        ],
    )(sp, hp, dinv, b.reshape(1, F), g.reshape(1, F), be.reshape(1, F), res,
      fcW1, fcb1.reshape(1, F), fcW2, fcb2.reshape(1, D))


def kernel(x, edge_index, W1, b1, W2, b2, W3, b3, g1, be1, g2, be2, g3, be3,
           fcW1, fcb1, fcW2, fcb2):
    N, F = x.shape
    E = edge_index.shape[1]

    ei = edge_index.reshape(2 * E)
    degp = _deg_kernel(N, E)(ei, jnp.zeros((N,), jnp.float32)).reshape(_NC, N)
    hp1, dinv = _input_proj(x, W1, degp)

    scat = _scatter_kernel(N, F, E)
    sp1 = scat(hp1, ei)
    z1, hp2 = _layer_tail(sp1, hp1, dinv, b1, g1, be1, None, W2)
    sp2 = scat(hp2, ei)
    z2, hp3 = _layer_tail(sp2, hp2, dinv, b2, g2, be2, z1, W3)
    sp3 = scat(hp3, ei)
    return _head(sp3, hp3, dinv, b3, g3, be3, z2, fcW1, fcb1, fcW2, fcb2)


# speedup vs baseline: 23.3790x; 1.0000x over previous
import functools

import jax
import jax.numpy as jnp
from jax import lax
from jax.experimental import pallas as pl
from jax.experimental.pallas import tpu as pltpu
from jax.experimental.pallas import tpu_sc as plsc

_NC = 2
_NS = 16
_K = 128
_R = 1000
_EPS = 1e-5


def _leaky(v):
    return jnp.where(v >= 0, v, 0.01 * v)


@functools.lru_cache(maxsize=None)
def _deg_kernel(N, E):
    NW = _NC * _NS
    epw = E // NW
    nb = epw // _K
    rem = epw - nb * _K
    assert E % NW == 0 and nb % 2 == 0 and nb >= 4
    assert rem % 8 == 0
    mesh = plsc.VectorSubcoreMesh(core_axis_name="c", subcore_axis_name="s")

    @functools.partial(
        pl.kernel,
        mesh=mesh,
        out_type=jax.ShapeDtypeStruct((_NC * N,), jnp.float32),
        scratch_types=[
            pltpu.VMEM((_K,), jnp.int32),
            pltpu.VMEM((_K,), jnp.int32),
            pltpu.VMEM((max(rem, 8),), jnp.int32),
            pltpu.VMEM((_K,), jnp.float32),
            pltpu.VMEM((N,), jnp.float32),
            pltpu.VMEM_SHARED((N,), jnp.float32),
            pltpu.SemaphoreType.DMA,
            pltpu.SemaphoreType.DMA,
        ],
    )
    def deg(ei, zeros, out, idx0, idx1, idxt, ones, stage, acc, sem0, sem1):
        c = lax.axis_index("c")
        s = lax.axis_index("s")
        base = E + (c * _NS + s) * epw

        @pl.when(s == 0)
        def _():
            pltpu.sync_copy(zeros, stage)
            pltpu.sync_copy(stage, acc)

        for j in range(_K // 16):
            ones[pl.ds(16 * j, 16)] = jnp.ones((16,), jnp.float32)
        plsc.subcore_barrier()

        def load(i, ref, sem):
            off = pl.multiple_of(base + i * _K, 8)
            pltpu.async_copy(ei.at[pl.ds(off, _K)], ref, sem)

        def scat(ref, sem):
            pltpu.make_async_copy(ei.at[pl.ds(0, _K)], ref, sem).wait()
            pltpu.sync_copy(ones, acc.at[ref], add=True)

        load(0, idx0, sem0)

        def pair(pi, carry):
            i = 2 * pi
            load(i + 1, idx1, sem1)
            scat(idx0, sem0)
            load(i + 2, idx0, sem0)
            scat(idx1, sem1)
            return carry

        lax.fori_loop(0, nb // 2 - 1, pair, 0)
        load(nb - 1, idx1, sem1)
        scat(idx0, sem0)
        scat(idx1, sem1)
        if rem:
            roff = pl.multiple_of(base + nb * _K, 8)
            pltpu.sync_copy(ei.at[pl.ds(roff, rem)], idxt)
            pltpu.sync_copy(ones.at[pl.ds(0, rem)], acc.at[idxt], add=True)

        plsc.subcore_barrier()

        @pl.when(s == 0)
        def _():
            coff = pl.multiple_of(c * N, 8)
            pltpu.sync_copy(acc, stage)
            pltpu.sync_copy(stage, out.at[pl.ds(coff, N)])

    return deg


@functools.lru_cache(maxsize=None)
def _scatter_kernel(N, F, E):
    NW = _NC * _NS
    epw = E // NW
    nb = epw // _K
    rem = epw - nb * _K
    assert E % NW == 0 and nb >= 9 and (nb - 3) % 3 == 0
    assert rem % 8 == 0
    rpt = (-(-N // _NS) + 15) // 16 * 16
    rpt_last = N - (_NS - 1) * rpt
    assert rpt_last > 0
    ntriples = (nb - 3) // 3 - 1
    mesh = plsc.VectorSubcoreMesh(core_axis_name="c", subcore_axis_name="s")

    @functools.partial(
        pl.kernel,
        mesh=mesh,
        out_type=jax.ShapeDtypeStruct((_NC, N, F), jnp.float32),
        scratch_types=[
            pltpu.VMEM((_K,), jnp.int32),
            pltpu.VMEM((_K,), jnp.int32),
            pltpu.VMEM((_K,), jnp.int32),
            pltpu.VMEM((_K,), jnp.int32),
            pltpu.VMEM((_K,), jnp.int32),
            pltpu.VMEM((_K,), jnp.int32),
            pltpu.VMEM((max(rem, 8),), jnp.int32),
            pltpu.VMEM((_K, F), jnp.float32),
            pltpu.VMEM((_K, F), jnp.float32),
            pltpu.VMEM((_K, F), jnp.float32),
            pltpu.VMEM_SHARED((N, F), jnp.float32),
            pltpu.SemaphoreType.DMA,
            pltpu.SemaphoreType.DMA,
            pltpu.SemaphoreType.DMA,
            pltpu.SemaphoreType.DMA,
            pltpu.SemaphoreType.DMA,
            pltpu.SemaphoreType.DMA,
            pltpu.SemaphoreType.DMA,
            pltpu.SemaphoreType.DMA,
            pltpu.SemaphoreType.DMA,
            pltpu.SemaphoreType.DMA,
            pltpu.SemaphoreType.DMA,
            pltpu.SemaphoreType.DMA,
        ],
    )
    def scatter(hp, ei, out, si0, si1, si2, di0, di1, di2, dit,
                rows0, rows1, rows2, acc,
                ss0, ss1, ss2, sd0, sd1, sd2, sg0, sg1, sg2, sc0, sc1, sc2):
        c = lax.axis_index("c")
        s = lax.axis_index("s")
        base = (c * _NS + s) * epw

        @pl.when(s < _NS - 1)
        def _():
            off = pl.multiple_of(s * rpt, 16)
            pltpu.sync_copy(hp.at[pl.ds(off, rpt)], acc.at[pl.ds(off, rpt)])

        @pl.when(s == _NS - 1)
        def _():
            off = (_NS - 1) * rpt
            pltpu.sync_copy(hp.at[pl.ds(off, rpt_last)],
                            acc.at[pl.ds(off, rpt_last)])

        plsc.subcore_barrier()

        def srcload(i, ref, sem):
            off = pl.multiple_of(base + i * _K, 8)
            pltpu.async_copy(ei.at[pl.ds(off, _K)], ref, sem)

        def dstload(i, ref, sem):
            off = pl.multiple_of(E + base + i * _K, 8)
            pltpu.async_copy(ei.at[pl.ds(off, _K)], ref, sem)

        def iwait(ref, sem):
            pltpu.make_async_copy(ei.at[pl.ds(0, _K)], ref, sem).wait()

        def gwait(rref, sem):
            pltpu.make_async_copy(hp.at[pl.ds(0, _K)], rref, sem).wait()

        def scwait(rref, iref, sem):
            pltpu.make_async_copy(rref, acc.at[iref], sem).wait()

        slots = ((si0, di0, rows0, ss0, sd0, sg0, sc0),
                 (si1, di1, rows1, ss1, sd1, sg1, sc1),
                 (si2, di2, rows2, ss2, sd2, sg2, sc2))

        def half(j, A, B, ld_src, ld_dst, do_scwait):
            siA, diA, rA, ssA, sdA, sgA, scA = A
            siB, diB, rB, ssB, sdB, sgB, scB = B
            gwait(rA, sgA)
            iwait(siB, ssB)
            if do_scwait:
                scwait(rB, diB, scB)
            pltpu.async_copy(hp.at[siB], rB, sgB)
            if ld_src:
                srcload(j + 3, siA, ssA)
            if ld_dst:
                dstload(j + 1, diB, sdB)
            iwait(diA, sdA)
            pltpu.async_copy(rA, acc.at[diA], scA, add=True)

        srcload(0, si0, ss0)
        dstload(0, di0, sd0)
        iwait(si0, ss0)
        pltpu.async_copy(hp.at[si0], rows0, sg0)
        srcload(1, si1, ss1)
        srcload(2, si2, ss2)

        half(0, slots[0], slots[1], True, True, False)
        half(1, slots[1], slots[2], True, True, False)

        def triple(t, carry):
            j = 3 * t + 2
            half(j, slots[2], slots[0], True, True, True)
            half(j + 1, slots[0], slots[1], True, True, True)
            half(j + 2, slots[1], slots[2], True, True, True)
            return carry

        lax.fori_loop(0, ntriples, triple, 0)
        half(nb - 4, slots[2], slots[0], True, True, True)
        half(nb - 3, slots[0], slots[1], False, True, True)
        half(nb - 2, slots[1], slots[2], False, True, True)
        gwait(rows2, sg2)
        iwait(di2, sd2)
        pltpu.sync_copy(rows2, acc.at[di2], add=True)
        scwait(rows0, di0, sc0)
        scwait(rows1, di1, sc1)

        if rem:
            roff = pl.multiple_of(base + nb * _K, 8)
            pltpu.sync_copy(ei.at[pl.ds(roff, rem)], si0.at[pl.ds(0, rem)])
            pltpu.sync_copy(ei.at[pl.ds(E + roff, rem)], dit)
            pltpu.sync_copy(hp.at[si0.at[pl.ds(0, rem)]],
                            rows0.at[pl.ds(0, rem)])
            pltpu.sync_copy(rows0.at[pl.ds(0, rem)], acc.at[dit], add=True)

        plsc.subcore_barrier()

        @pl.when(s < _NS - 1)
        def _():
            off = pl.multiple_of(s * rpt, 16)
            pltpu.sync_copy(acc.at[pl.ds(off, rpt)],
                            out.at[c, pl.ds(off, rpt)])

        @pl.when(s == _NS - 1)
        def _():
            off = (_NS - 1) * rpt
            pltpu.sync_copy(acc.at[pl.ds(off, rpt_last)],
                            out.at[c, pl.ds(off, rpt_last)])

    return scatter


def _input_proj(x, W1, degp):
    N, F = x.shape
    nbk = N // _R

    def body(x_ref, w_ref, d_ref, hp_ref, dinv_ref):
        deg = d_ref[0] + d_ref[1] + 1.0
        dv = lax.rsqrt(deg)
        dinv_ref[...] = dv
        hp_ref[...] = dv * jnp.dot(x_ref[...], w_ref[...],
                                   preferred_element_type=jnp.float32)

    return pl.pallas_call(
        body,
        grid=(nbk,),
        in_specs=[
            pl.BlockSpec((_R, F), lambda i: (i, 0)),
            pl.BlockSpec((F, F), lambda i: (0, 0)),
            pl.BlockSpec((2, _R, 1), lambda i: (0, i, 0)),
        ],
        out_specs=[
            pl.BlockSpec((_R, F), lambda i: (i, 0)),
            pl.BlockSpec((_R, 1), lambda i: (i, 0)),
        ],
        out_shape=[
            jax.ShapeDtypeStruct((N, F), jnp.float32),
            jax.ShapeDtypeStruct((N, 1), jnp.float32),
        ],
    )(x, W1, degp.reshape(_NC, N, 1))


def _layer_tail(sp, hp, dinv, b, g, be, res, Wn):
    N, F = hp.shape
    nbk = N // _R
    has_res = res is not None

    def body(*refs):
        if has_res:
            (sp_ref, hp_ref, dinv_ref, b_ref, g_ref, be_ref, res_ref, w_ref,
             z_ref, hpn_ref, y_s, ssum, ssq) = refs
        else:
            (sp_ref, hp_ref, dinv_ref, b_ref, g_ref, be_ref, w_ref,
             z_ref, hpn_ref, y_s, ssum, ssq) = refs
        p = pl.program_id(0)
        i = pl.program_id(1)

        @pl.when(p == 0)
        def _():
            y = ((sp_ref[0] + sp_ref[1] - hp_ref[...])
                 * dinv_ref[...] + b_ref[...])
            y_s[pl.ds(i * _R, _R), :] = y

            @pl.when(i == 0)
            def _():
                ssum[...] = jnp.zeros_like(ssum)
                ssq[...] = jnp.zeros_like(ssq)

            ssum[...] += jnp.sum(y, axis=0, keepdims=True)
            ssq[...] += jnp.sum(y * y, axis=0, keepdims=True)

        @pl.when(p == 1)
        def _():
            m = ssum[...] / N
            v = ssq[...] / N - m * m
            y = y_s[pl.ds(i * _R, _R), :]
            yn = (y - m) * lax.rsqrt(v + _EPS) * g_ref[...] + be_ref[...]
            z = _leaky(yn)
            if has_res:
                z = z + res_ref[...]
            z_ref[...] = z
            hpn_ref[...] = dinv_ref[...] * jnp.dot(
                z, w_ref[...], preferred_element_type=jnp.float32)

    frozen = lambda p, i: (i * (1 - p) + (nbk - 1) * p, 0)
    sp_spec = pl.BlockSpec((_NC, _R, F), lambda p, i: (0,) + frozen(p, i))
    in_specs = [
        sp_spec,
        pl.BlockSpec((_R, F), frozen),
        pl.BlockSpec((_R, 1), lambda p, i: (i, 0)),
        pl.BlockSpec((1, F), lambda p, i: (0, 0)),
        pl.BlockSpec((1, F), lambda p, i: (0, 0)),
        pl.BlockSpec((1, F), lambda p, i: (0, 0)),
    ]
    args = [sp, hp, dinv, b.reshape(1, F), g.reshape(1, F), be.reshape(1, F)]
    if has_res:
        in_specs.append(pl.BlockSpec((_R, F), lambda p, i: (i * p, 0)))
        args.append(res)
    in_specs.append(pl.BlockSpec((F, F), lambda p, i: (0, 0)))
    args.append(Wn)

    return pl.pallas_call(
        body,
        grid=(2, nbk),
        in_specs=in_specs,
        out_specs=[
            pl.BlockSpec((_R, F), lambda p, i: (i * p, 0)),
            pl.BlockSpec((_R, F), lambda p, i: (i * p, 0)),
        ],
        out_shape=[
            jax.ShapeDtypeStruct((N, F), jnp.float32),
            jax.ShapeDtypeStruct((N, F), jnp.float32),
        ],
        scratch_shapes=[
            pltpu.VMEM((N, F), jnp.float32),
            pltpu.VMEM((1, F), jnp.float32),
            pltpu.VMEM((1, F), jnp.float32),
        ],
    )(*args)


def _head(sp, hp, dinv, b, g, be, res, fcW1, fcb1, fcW2, fcb2):
    N, F = hp.shape
    D = fcW2.shape[1]
    nbk = N // _R

    def body(sp_ref, hp_ref, dinv_ref, b_ref, g_ref, be_ref, res_ref,
             w1_ref, b1_ref, w2_ref, b2_ref, out_ref, y_s, ssum, ssq, zsum):
        p = pl.program_id(0)
        i = pl.program_id(1)

        @pl.when(p == 0)
        def _():
            y = ((sp_ref[0] + sp_ref[1] - hp_ref[...])
                 * dinv_ref[...] + b_ref[...])
            y_s[pl.ds(i * _R, _R), :] = y

            @pl.when(i == 0)
            def _():
                ssum[...] = jnp.zeros_like(ssum)
                ssq[...] = jnp.zeros_like(ssq)

            ssum[...] += jnp.sum(y, axis=0, keepdims=True)
            ssq[...] += jnp.sum(y * y, axis=0, keepdims=True)

        @pl.when(p == 1)
        def _():
            m = ssum[...] / N
            v = ssq[...] / N - m * m
            y = y_s[pl.ds(i * _R, _R), :]
            yn = (y - m) * lax.rsqrt(v + _EPS) * g_ref[...] + be_ref[...]
            z = _leaky(yn) + res_ref[...]

            @pl.when(i == 0)
            def _():
                zsum[...] = jnp.zeros_like(zsum)

            zsum[...] += jnp.sum(z, axis=0, keepdims=True)

            @pl.when(i == nbk - 1)
            def _():
                pooled = zsum[...] / N
                h1 = _leaky(jnp.dot(pooled, w1_ref[...],
                                    preferred_element_type=jnp.float32)
                            + b1_ref[...])
                o = jnp.dot(h1, w2_ref[...],
                            preferred_element_type=jnp.float32) + b2_ref[...]
                nrm = jnp.sqrt(jnp.sum(o * o, axis=1, keepdims=True))
                out_ref[...] = o / jnp.maximum(nrm, 1e-12)

    frozen = lambda p, i: (i * (1 - p) + (nbk - 1) * p, 0)
    return pl.pallas_call(
        body,
        grid=(2, nbk),
        in_specs=[
            pl.BlockSpec((_NC, _R, F), lambda p, i: (0,) + frozen(p, i)),
            pl.BlockSpec((_R, F), frozen),
            pl.BlockSpec((_R, 1), lambda p, i: (i, 0)),
            pl.BlockSpec((1, F), lambda p, i: (0, 0)),
            pl.BlockSpec((1, F), lambda p, i: (0, 0)),
            pl.BlockSpec((1, F), lambda p, i: (0, 0)),
            pl.BlockSpec((_R, F), lambda p, i: (i * p, 0)),
            pl.BlockSpec((F, F), lambda p, i: (0, 0)),
            pl.BlockSpec((1, F), lambda p, i: (0, 0)),
            pl.BlockSpec((F, D), lambda p, i: (0, 0)),
            pl.BlockSpec((1, D), lambda p, i: (0, 0)),
        ],
        out_specs=pl.BlockSpec((1, D), lambda p, i: (0, 0)),
        out_shape=jax.ShapeDtypeStruct((1, D), jnp.float32),
        scratch_shapes=[
            pltpu.VMEM((N, F), jnp.float32),
            pltpu.VMEM((1, F), jnp.float32),
            pltpu.VMEM((1, F), jnp.float32),
            pltpu.VMEM((1, F), jnp.float32),
        ],
    )(sp, hp, dinv, b.reshape(1, F), g.reshape(1, F), be.reshape(1, F), res,
      fcW1, fcb1.reshape(1, F), fcW2, fcb2.reshape(1, D))


def kernel(x, edge_index, W1, b1, W2, b2, W3, b3, g1, be1, g2, be2, g3, be3,
           fcW1, fcb1, fcW2, fcb2):
    N, F = x.shape
    E = edge_index.shape[1]

    ei = edge_index.reshape(2 * E)
    degp = _deg_kernel(N, E)(ei, jnp.zeros((N,), jnp.float32)).reshape(_NC, N)
    hp1, dinv = _input_proj(x, W1, degp)

    scat = _scatter_kernel(N, F, E)
    sp1 = scat(hp1, ei)
    z1, hp2 = _layer_tail(sp1, hp1, dinv, b1, g1, be1, None, W2)
    sp2 = scat(hp2, ei)
    z2, hp3 = _layer_tail(sp2, hp2, dinv, b2, g2, be2, z1, W3)
    sp3 = scat(hp3, ei)
    return _head(sp3, hp3, dinv, b3, g3, be3, z2, fcW1, fcb1, fcW2, fcb2)


# speedup vs baseline: 23.3846x; 1.0002x over previous
import functools

import jax
import jax.numpy as jnp
from jax import lax
from jax.experimental import pallas as pl
from jax.experimental.pallas import tpu as pltpu
from jax.experimental.pallas import tpu_sc as plsc

_NC = 2
_NS = 16
_K = 128
_R = 1000
_EPS = 1e-5


def _leaky(v):
    return jnp.where(v >= 0, v, 0.01 * v)


@functools.lru_cache(maxsize=None)
def _deg_kernel(N, E):
    NW = _NC * _NS
    epw = E // NW
    nb = epw // _K
    rem = epw - nb * _K
    assert E % NW == 0 and nb % 2 == 0 and nb >= 4
    assert rem % 8 == 0
    mesh = plsc.VectorSubcoreMesh(core_axis_name="c", subcore_axis_name="s")

    @functools.partial(
        pl.kernel,
        mesh=mesh,
        out_type=jax.ShapeDtypeStruct((_NC * N,), jnp.float32),
        scratch_types=[
            pltpu.VMEM((_K,), jnp.int32),
            pltpu.VMEM((_K,), jnp.int32),
            pltpu.VMEM((max(rem, 8),), jnp.int32),
            pltpu.VMEM((_K,), jnp.float32),
            pltpu.VMEM((N,), jnp.float32),
            pltpu.VMEM_SHARED((N,), jnp.float32),
            pltpu.SemaphoreType.DMA,
            pltpu.SemaphoreType.DMA,
        ],
    )
    def deg(ei, zeros, out, idx0, idx1, idxt, ones, stage, acc, sem0, sem1):
        c = lax.axis_index("c")
        s = lax.axis_index("s")
        base = E + (c * _NS + s) * epw

        @pl.when(s == 0)
        def _():
            pltpu.sync_copy(zeros, stage)
            pltpu.sync_copy(stage, acc)

        for j in range(_K // 16):
            ones[pl.ds(16 * j, 16)] = jnp.ones((16,), jnp.float32)
        plsc.subcore_barrier()

        def load(i, ref, sem):
            off = pl.multiple_of(base + i * _K, 8)
            pltpu.async_copy(ei.at[pl.ds(off, _K)], ref, sem)

        def scat(ref, sem):
            pltpu.make_async_copy(ei.at[pl.ds(0, _K)], ref, sem).wait()
            pltpu.sync_copy(ones, acc.at[ref], add=True)

        load(0, idx0, sem0)

        def pair(pi, carry):
            i = 2 * pi
            load(i + 1, idx1, sem1)
            scat(idx0, sem0)
            load(i + 2, idx0, sem0)
            scat(idx1, sem1)
            return carry

        lax.fori_loop(0, nb // 2 - 1, pair, 0)
        load(nb - 1, idx1, sem1)
        scat(idx0, sem0)
        scat(idx1, sem1)
        if rem:
            roff = pl.multiple_of(base + nb * _K, 8)
            pltpu.sync_copy(ei.at[pl.ds(roff, rem)], idxt)
            pltpu.sync_copy(ones.at[pl.ds(0, rem)], acc.at[idxt], add=True)

        plsc.subcore_barrier()

        @pl.when(s == 0)
        def _():
            coff = pl.multiple_of(c * N, 8)
            pltpu.sync_copy(acc, stage)
            pltpu.sync_copy(stage, out.at[pl.ds(coff, N)])

    return deg


@functools.lru_cache(maxsize=None)
def _scatter_kernel(N, F, E):
    NW = _NC * _NS
    epw = E // NW
    nb = epw // _K
    rem = epw - nb * _K
    assert E % NW == 0 and nb >= 9 and (nb - 3) % 3 == 0
    assert rem % 8 == 0
    rpt = (-(-N // _NS) + 15) // 16 * 16
    rpt_last = N - (_NS - 1) * rpt
    assert rpt_last > 0
    ntriples = (nb - 3) // 3 - 1
    mesh = plsc.VectorSubcoreMesh(core_axis_name="c", subcore_axis_name="s")

    @functools.partial(
        pl.kernel,
        mesh=mesh,
        out_type=jax.ShapeDtypeStruct((_NC, N, F), jnp.float32),
        scratch_types=[
            pltpu.VMEM((_K,), jnp.int32),
            pltpu.VMEM((_K,), jnp.int32),
            pltpu.VMEM((_K,), jnp.int32),
            pltpu.VMEM((_K,), jnp.int32),
            pltpu.VMEM((_K,), jnp.int32),
            pltpu.VMEM((_K,), jnp.int32),
            pltpu.VMEM((max(rem, 8),), jnp.int32),
            pltpu.VMEM((_K, F), jnp.float32),
            pltpu.VMEM((_K, F), jnp.float32),
            pltpu.VMEM((_K, F), jnp.float32),
            pltpu.VMEM_SHARED((N, F), jnp.float32),
            pltpu.SemaphoreType.DMA,
            pltpu.SemaphoreType.DMA,
            pltpu.SemaphoreType.DMA,
            pltpu.SemaphoreType.DMA,
            pltpu.SemaphoreType.DMA,
            pltpu.SemaphoreType.DMA,
            pltpu.SemaphoreType.DMA,
            pltpu.SemaphoreType.DMA,
            pltpu.SemaphoreType.DMA,
            pltpu.SemaphoreType.DMA,
            pltpu.SemaphoreType.DMA,
            pltpu.SemaphoreType.DMA,
        ],
    )
    def scatter(hp, ei, out, si0, si1, si2, di0, di1, di2, dit,
                rows0, rows1, rows2, acc,
                ss0, ss1, ss2, sd0, sd1, sd2, sg0, sg1, sg2, sc0, sc1, sc2):
        c = lax.axis_index("c")
        s = lax.axis_index("s")
        base = (c * _NS + s) * epw

        @pl.when(s < _NS - 1)
        def _():
            off = pl.multiple_of(s * rpt, 16)
            pltpu.sync_copy(hp.at[pl.ds(off, rpt)], acc.at[pl.ds(off, rpt)])

        @pl.when(s == _NS - 1)
        def _():
            off = (_NS - 1) * rpt
            pltpu.sync_copy(hp.at[pl.ds(off, rpt_last)],
                            acc.at[pl.ds(off, rpt_last)])

        plsc.subcore_barrier()

        def srcload(i, ref, sem):
            off = pl.multiple_of(base + i * _K, 8)
            pltpu.async_copy(ei.at[pl.ds(off, _K)], ref, sem)

        def dstload(i, ref, sem):
            off = pl.multiple_of(E + base + i * _K, 8)
            pltpu.async_copy(ei.at[pl.ds(off, _K)], ref, sem)

        def iwait(ref, sem):
            pltpu.make_async_copy(ei.at[pl.ds(0, _K)], ref, sem).wait()

        def gwait(rref, sem):
            pltpu.make_async_copy(hp.at[pl.ds(0, _K)], rref, sem).wait()

        def scwait(rref, iref, sem):
            pltpu.make_async_copy(rref, acc.at[iref], sem).wait()

        slots = ((si0, di0, rows0, ss0, sd0, sg0, sc0),
                 (si1, di1, rows1, ss1, sd1, sg1, sc1),
                 (si2, di2, rows2, ss2, sd2, sg2, sc2))

        def half(j, A, B, ld_src, ld_dst, do_scwait):
            siA, diA, rA, ssA, sdA, sgA, scA = A
            siB, diB, rB, ssB, sdB, sgB, scB = B
            gwait(rA, sgA)
            iwait(siB, ssB)
            if do_scwait:
                scwait(rB, diB, scB)
            pltpu.async_copy(hp.at[siB], rB, sgB)
            if ld_src:
                srcload(j + 3, siA, ssA)
            if ld_dst:
                dstload(j + 1, diB, sdB)
            iwait(diA, sdA)
            pltpu.async_copy(rA, acc.at[diA], scA, add=True)

        srcload(0, si0, ss0)
        dstload(0, di0, sd0)
        iwait(si0, ss0)
        pltpu.async_copy(hp.at[si0], rows0, sg0)
        srcload(1, si1, ss1)
        srcload(2, si2, ss2)

        half(0, slots[0], slots[1], True, True, False)
        half(1, slots[1], slots[2], True, True, False)

        def triple(t, carry):
            j = 3 * t + 2
            half(j, slots[2], slots[0], True, True, True)
            half(j + 1, slots[0], slots[1], True, True, True)
            half(j + 2, slots[1], slots[2], True, True, True)
            return carry

        lax.fori_loop(0, ntriples, triple, 0)
        half(nb - 4, slots[2], slots[0], True, True, True)
        half(nb - 3, slots[0], slots[1], False, True, True)
        half(nb - 2, slots[1], slots[2], False, True, True)
        gwait(rows2, sg2)
        iwait(di2, sd2)
        pltpu.sync_copy(rows2, acc.at[di2], add=True)
        scwait(rows0, di0, sc0)
        scwait(rows1, di1, sc1)

        if rem:
            roff = pl.multiple_of(base + nb * _K, 8)
            pltpu.sync_copy(ei.at[pl.ds(roff, rem)], si0.at[pl.ds(0, rem)])
            pltpu.sync_copy(ei.at[pl.ds(E + roff, rem)], dit)
            pltpu.sync_copy(hp.at[si0.at[pl.ds(0, rem)]],
                            rows0.at[pl.ds(0, rem)])
            pltpu.sync_copy(rows0.at[pl.ds(0, rem)], acc.at[dit], add=True)

        plsc.subcore_barrier()

        @pl.when(s < _NS - 1)
        def _():
            off = pl.multiple_of(s * rpt, 16)
            pltpu.sync_copy(acc.at[pl.ds(off, rpt)],
                            out.at[c, pl.ds(off, rpt)])

        @pl.when(s == _NS - 1)
        def _():
            off = (_NS - 1) * rpt
            pltpu.sync_copy(acc.at[pl.ds(off, rpt_last)],
                            out.at[c, pl.ds(off, rpt_last)])

    return scatter


def _matmul(x, W1):
    N, F = x.shape
    nbk = N // _R

    def body(x_ref, w_ref, o_ref):
        o_ref[...] = jnp.dot(x_ref[...], w_ref[...],
                             preferred_element_type=jnp.float32)

    return pl.pallas_call(
        body,
        grid=(nbk,),
        in_specs=[
            pl.BlockSpec((_R, F), lambda i: (i, 0)),
            pl.BlockSpec((F, F), lambda i: (0, 0)),
        ],
        out_specs=pl.BlockSpec((_R, F), lambda i: (i, 0)),
        out_shape=jax.ShapeDtypeStruct((N, F), jnp.float32),
    )(x, W1)


def _scale(xw, degp):
    N, F = xw.shape
    nbk = N // _R

    def body(xw_ref, d_ref, hp_ref, dinv_ref):
        deg = d_ref[0] + d_ref[1] + 1.0
        dv = lax.rsqrt(deg)
        dinv_ref[...] = dv
        hp_ref[...] = dv * xw_ref[...]

    return pl.pallas_call(
        body,
        grid=(nbk,),
        in_specs=[
            pl.BlockSpec((_R, F), lambda i: (i, 0)),
            pl.BlockSpec((2, _R, 1), lambda i: (0, i, 0)),
        ],
        out_specs=[
            pl.BlockSpec((_R, F), lambda i: (i, 0)),
            pl.BlockSpec((_R, 1), lambda i: (i, 0)),
        ],
        out_shape=[
            jax.ShapeDtypeStruct((N, F), jnp.float32),
            jax.ShapeDtypeStruct((N, 1), jnp.float32),
        ],
    )(xw, degp.reshape(_NC, N, 1))


def _layer_tail(sp, hp, dinv, b, g, be, res, Wn):
    N, F = hp.shape
    nbk = N // _R
    has_res = res is not None

    def body(*refs):
        if has_res:
            (sp_ref, hp_ref, dinv_ref, b_ref, g_ref, be_ref, res_ref, w_ref,
             z_ref, hpn_ref, y_s, ssum, ssq) = refs
        else:
            (sp_ref, hp_ref, dinv_ref, b_ref, g_ref, be_ref, w_ref,
             z_ref, hpn_ref, y_s, ssum, ssq) = refs
        p = pl.program_id(0)
        i = pl.program_id(1)

        @pl.when(p == 0)
        def _():
            y = ((sp_ref[0] + sp_ref[1] - hp_ref[...])
                 * dinv_ref[...] + b_ref[...])
            y_s[pl.ds(i * _R, _R), :] = y

            @pl.when(i == 0)
            def _():
                ssum[...] = jnp.zeros_like(ssum)
                ssq[...] = jnp.zeros_like(ssq)

            ssum[...] += jnp.sum(y, axis=0, keepdims=True)
            ssq[...] += jnp.sum(y * y, axis=0, keepdims=True)

        @pl.when(p == 1)
        def _():
            m = ssum[...] / N
            v = ssq[...] / N - m * m
            y = y_s[pl.ds(i * _R, _R), :]
            yn = (y - m) * lax.rsqrt(v + _EPS) * g_ref[...] + be_ref[...]
            z = _leaky(yn)
            if has_res:
                z = z + res_ref[...]
            z_ref[...] = z
            hpn_ref[...] = dinv_ref[...] * jnp.dot(
                z, w_ref[...], preferred_element_type=jnp.float32)

    frozen = lambda p, i: (i * (1 - p) + (nbk - 1) * p, 0)
    sp_spec = pl.BlockSpec((_NC, _R, F), lambda p, i: (0,) + frozen(p, i))
    in_specs = [
        sp_spec,
        pl.BlockSpec((_R, F), frozen),
        pl.BlockSpec((_R, 1), lambda p, i: (i, 0)),
        pl.BlockSpec((1, F), lambda p, i: (0, 0)),
        pl.BlockSpec((1, F), lambda p, i: (0, 0)),
        pl.BlockSpec((1, F), lambda p, i: (0, 0)),
    ]
    args = [sp, hp, dinv, b.reshape(1, F), g.reshape(1, F), be.reshape(1, F)]
    if has_res:
        in_specs.append(pl.BlockSpec((_R, F), lambda p, i: (i * p, 0)))
        args.append(res)
    in_specs.append(pl.BlockSpec((F, F), lambda p, i: (0, 0)))
    args.append(Wn)

    return pl.pallas_call(
        body,
        grid=(2, nbk),
        in_specs=in_specs,
        out_specs=[
            pl.BlockSpec((_R, F), lambda p, i: (i * p, 0)),
            pl.BlockSpec((_R, F), lambda p, i: (i * p, 0)),
        ],
        out_shape=[
            jax.ShapeDtypeStruct((N, F), jnp.float32),
            jax.ShapeDtypeStruct((N, F), jnp.float32),
        ],
        scratch_shapes=[
            pltpu.VMEM((N, F), jnp.float32),
            pltpu.VMEM((1, F), jnp.float32),
            pltpu.VMEM((1, F), jnp.float32),
        ],
    )(*args)


def _head(sp, hp, dinv, b, g, be, res, fcW1, fcb1, fcW2, fcb2):
    N, F = hp.shape
    D = fcW2.shape[1]
    nbk = N // _R

    def body(sp_ref, hp_ref, dinv_ref, b_ref, g_ref, be_ref, res_ref,
             w1_ref, b1_ref, w2_ref, b2_ref, out_ref, y_s, ssum, ssq, zsum):
        p = pl.program_id(0)
        i = pl.program_id(1)

        @pl.when(p == 0)
        def _():
            y = ((sp_ref[0] + sp_ref[1] - hp_ref[...])
                 * dinv_ref[...] + b_ref[...])
            y_s[pl.ds(i * _R, _R), :] = y

            @pl.when(i == 0)
            def _():
                ssum[...] = jnp.zeros_like(ssum)
                ssq[...] = jnp.zeros_like(ssq)

            ssum[...] += jnp.sum(y, axis=0, keepdims=True)
            ssq[...] += jnp.sum(y * y, axis=0, keepdims=True)

        @pl.when(p == 1)
        def _():
            m = ssum[...] / N
            v = ssq[...] / N - m * m
            y = y_s[pl.ds(i * _R, _R), :]
            yn = (y - m) * lax.rsqrt(v + _EPS) * g_ref[...] + be_ref[...]
            z = _leaky(yn) + res_ref[...]

            @pl.when(i == 0)
            def _():
                zsum[...] = jnp.zeros_like(zsum)

            zsum[...] += jnp.sum(z, axis=0, keepdims=True)

            @pl.when(i == nbk - 1)
            def _():
                pooled = zsum[...] / N
                h1 = _leaky(jnp.dot(pooled, w1_ref[...],
                                    preferred_element_type=jnp.float32)
                            + b1_ref[...])
                o = jnp.dot(h1, w2_ref[...],
                            preferred_element_type=jnp.float32) + b2_ref[...]
                nrm = jnp.sqrt(jnp.sum(o * o, axis=1, keepdims=True))
                out_ref[...] = o / jnp.maximum(nrm, 1e-12)

    frozen = lambda p, i: (i * (1 - p) + (nbk - 1) * p, 0)
    return pl.pallas_call(
        body,
        grid=(2, nbk),
        in_specs=[
            pl.BlockSpec((_NC, _R, F), lambda p, i: (0,) + frozen(p, i)),
            pl.BlockSpec((_R, F), frozen),
            pl.BlockSpec((_R, 1), lambda p, i: (i, 0)),
            pl.BlockSpec((1, F), lambda p, i: (0, 0)),
            pl.BlockSpec((1, F), lambda p, i: (0, 0)),
            pl.BlockSpec((1, F), lambda p, i: (0, 0)),
            pl.BlockSpec((_R, F), lambda p, i: (i * p, 0)),
            pl.BlockSpec((F, F), lambda p, i: (0, 0)),
            pl.BlockSpec((1, F), lambda p, i: (0, 0)),
            pl.BlockSpec((F, D), lambda p, i: (0, 0)),
            pl.BlockSpec((1, D), lambda p, i: (0, 0)),
        ],
        out_specs=pl.BlockSpec((1, D), lambda p, i: (0, 0)),
        out_shape=jax.ShapeDtypeStruct((1, D), jnp.float32),
        scratch_shapes=[
            pltpu.VMEM((N, F), jnp.float32),
            pltpu.VMEM((1, F), jnp.float32),
            pltpu.VMEM((1, F), jnp.float32),
            pltpu.VMEM((1, F), jnp.float32),
        ],
    )(sp, hp, dinv, b.reshape(1, F), g.reshape(1, F), be.reshape(1, F), res,
      fcW1, fcb1.reshape(1, F), fcW2, fcb2.reshape(1, D))


def kernel(x, edge_index, W1, b1, W2, b2, W3, b3, g1, be1, g2, be2, g3, be3,
           fcW1, fcb1, fcW2, fcb2):
    N, F = x.shape
    E = edge_index.shape[1]

    ei = edge_index.reshape(2 * E)
    degp = _deg_kernel(N, E)(ei, jnp.zeros((N,), jnp.float32)).reshape(_NC, N)
    xw = _matmul(x, W1)
    hp1, dinv = _scale(xw, degp)

    scat = _scatter_kernel(N, F, E)
    sp1 = scat(hp1, ei)
    z1, hp2 = _layer_tail(sp1, hp1, dinv, b1, g1, be1, None, W2)
    sp2 = scat(hp2, ei)
    z2, hp3 = _layer_tail(sp2, hp2, dinv, b2, g2, be2, z1, W3)
    sp3 = scat(hp3, ei)
    return _head(sp3, hp3, dinv, b3, g3, be3, z2, fcW1, fcb1, fcW2, fcb2)


# speedup vs baseline: 24.1067x; 1.0309x over previous
import functools

import jax
import jax.numpy as jnp
from jax import lax
from jax.experimental import pallas as pl
from jax.experimental.pallas import tpu as pltpu
from jax.experimental.pallas import tpu_sc as plsc

_NC = 2
_NS = 16
_K = 128
_R = 2000
_EPS = 1e-5


def _leaky(v):
    return jnp.where(v >= 0, v, 0.01 * v)


@functools.lru_cache(maxsize=None)
def _deg_kernel(N, E):
    NW = _NC * _NS
    epw = E // NW
    nb = epw // _K
    rem = epw - nb * _K
    assert E % NW == 0 and nb % 2 == 0 and nb >= 4
    assert rem % 8 == 0
    mesh = plsc.VectorSubcoreMesh(core_axis_name="c", subcore_axis_name="s")

    @functools.partial(
        pl.kernel,
        mesh=mesh,
        out_type=jax.ShapeDtypeStruct((_NC * N,), jnp.float32),
        scratch_types=[
            pltpu.VMEM((_K,), jnp.int32),
            pltpu.VMEM((_K,), jnp.int32),
            pltpu.VMEM((max(rem, 8),), jnp.int32),
            pltpu.VMEM((_K,), jnp.float32),
            pltpu.VMEM((N,), jnp.float32),
            pltpu.VMEM_SHARED((N,), jnp.float32),
            pltpu.SemaphoreType.DMA,
            pltpu.SemaphoreType.DMA,
        ],
    )
    def deg(ei, zeros, out, idx0, idx1, idxt, ones, stage, acc, sem0, sem1):
        c = lax.axis_index("c")
        s = lax.axis_index("s")
        base = E + (c * _NS + s) * epw

        @pl.when(s == 0)
        def _():
            pltpu.sync_copy(zeros, stage)
            pltpu.sync_copy(stage, acc)

        for j in range(_K // 16):
            ones[pl.ds(16 * j, 16)] = jnp.ones((16,), jnp.float32)
        plsc.subcore_barrier()

        def load(i, ref, sem):
            off = pl.multiple_of(base + i * _K, 8)
            pltpu.async_copy(ei.at[pl.ds(off, _K)], ref, sem)

        def scat(ref, sem):
            pltpu.make_async_copy(ei.at[pl.ds(0, _K)], ref, sem).wait()
            pltpu.sync_copy(ones, acc.at[ref], add=True)

        load(0, idx0, sem0)

        def pair(pi, carry):
            i = 2 * pi
            load(i + 1, idx1, sem1)
            scat(idx0, sem0)
            load(i + 2, idx0, sem0)
            scat(idx1, sem1)
            return carry

        lax.fori_loop(0, nb // 2 - 1, pair, 0)
        load(nb - 1, idx1, sem1)
        scat(idx0, sem0)
        scat(idx1, sem1)
        if rem:
            roff = pl.multiple_of(base + nb * _K, 8)
            pltpu.sync_copy(ei.at[pl.ds(roff, rem)], idxt)
            pltpu.sync_copy(ones.at[pl.ds(0, rem)], acc.at[idxt], add=True)

        plsc.subcore_barrier()

        @pl.when(s == 0)
        def _():
            coff = pl.multiple_of(c * N, 8)
            pltpu.sync_copy(acc, stage)
            pltpu.sync_copy(stage, out.at[pl.ds(coff, N)])

    return deg


@functools.lru_cache(maxsize=None)
def _scatter_kernel(N, F, E):
    NW = _NC * _NS
    epw = E // NW
    nb = epw // _K
    rem = epw - nb * _K
    assert E % NW == 0 and nb >= 9 and (nb - 3) % 3 == 0
    assert rem % 8 == 0
    rpt = (-(-N // _NS) + 15) // 16 * 16
    rpt_last = N - (_NS - 1) * rpt
    assert rpt_last > 0
    ntriples = (nb - 3) // 3 - 1
    mesh = plsc.VectorSubcoreMesh(core_axis_name="c", subcore_axis_name="s")

    @functools.partial(
        pl.kernel,
        mesh=mesh,
        out_type=jax.ShapeDtypeStruct((_NC, N, F), jnp.float32),
        scratch_types=[
            pltpu.VMEM((_K,), jnp.int32),
            pltpu.VMEM((_K,), jnp.int32),
            pltpu.VMEM((_K,), jnp.int32),
            pltpu.VMEM((_K,), jnp.int32),
            pltpu.VMEM((_K,), jnp.int32),
            pltpu.VMEM((_K,), jnp.int32),
            pltpu.VMEM((max(rem, 8),), jnp.int32),
            pltpu.VMEM((_K, F), jnp.float32),
            pltpu.VMEM((_K, F), jnp.float32),
            pltpu.VMEM((_K, F), jnp.float32),
            pltpu.VMEM_SHARED((N, F), jnp.float32),
            pltpu.SemaphoreType.DMA,
            pltpu.SemaphoreType.DMA,
            pltpu.SemaphoreType.DMA,
            pltpu.SemaphoreType.DMA,
            pltpu.SemaphoreType.DMA,
            pltpu.SemaphoreType.DMA,
            pltpu.SemaphoreType.DMA,
            pltpu.SemaphoreType.DMA,
            pltpu.SemaphoreType.DMA,
            pltpu.SemaphoreType.DMA,
            pltpu.SemaphoreType.DMA,
            pltpu.SemaphoreType.DMA,
        ],
    )
    def scatter(hp, ei, out, si0, si1, si2, di0, di1, di2, dit,
                rows0, rows1, rows2, acc,
                ss0, ss1, ss2, sd0, sd1, sd2, sg0, sg1, sg2, sc0, sc1, sc2):
        c = lax.axis_index("c")
        s = lax.axis_index("s")
        base = (c * _NS + s) * epw

        @pl.when(s < _NS - 1)
        def _():
            off = pl.multiple_of(s * rpt, 16)
            pltpu.sync_copy(hp.at[pl.ds(off, rpt)], acc.at[pl.ds(off, rpt)])

        @pl.when(s == _NS - 1)
        def _():
            off = (_NS - 1) * rpt
            pltpu.sync_copy(hp.at[pl.ds(off, rpt_last)],
                            acc.at[pl.ds(off, rpt_last)])

        plsc.subcore_barrier()

        def srcload(i, ref, sem):
            off = pl.multiple_of(base + i * _K, 8)
            pltpu.async_copy(ei.at[pl.ds(off, _K)], ref, sem)

        def dstload(i, ref, sem):
            off = pl.multiple_of(E + base + i * _K, 8)
            pltpu.async_copy(ei.at[pl.ds(off, _K)], ref, sem)

        def iwait(ref, sem):
            pltpu.make_async_copy(ei.at[pl.ds(0, _K)], ref, sem).wait()

        def gwait(rref, sem):
            pltpu.make_async_copy(hp.at[pl.ds(0, _K)], rref, sem).wait()

        def scwait(rref, iref, sem):
            pltpu.make_async_copy(rref, acc.at[iref], sem).wait()

        slots = ((si0, di0, rows0, ss0, sd0, sg0, sc0),
                 (si1, di1, rows1, ss1, sd1, sg1, sc1),
                 (si2, di2, rows2, ss2, sd2, sg2, sc2))

        def half(j, A, B, ld_src, ld_dst, do_scwait):
            siA, diA, rA, ssA, sdA, sgA, scA = A
            siB, diB, rB, ssB, sdB, sgB, scB = B
            gwait(rA, sgA)
            iwait(siB, ssB)
            if do_scwait:
                scwait(rB, diB, scB)
            pltpu.async_copy(hp.at[siB], rB, sgB)
            if ld_src:
                srcload(j + 3, siA, ssA)
            if ld_dst:
                dstload(j + 1, diB, sdB)
            iwait(diA, sdA)
            pltpu.async_copy(rA, acc.at[diA], scA, add=True)

        srcload(0, si0, ss0)
        dstload(0, di0, sd0)
        iwait(si0, ss0)
        pltpu.async_copy(hp.at[si0], rows0, sg0)
        srcload(1, si1, ss1)
        srcload(2, si2, ss2)

        half(0, slots[0], slots[1], True, True, False)
        half(1, slots[1], slots[2], True, True, False)

        def triple(t, carry):
            j = 3 * t + 2
            half(j, slots[2], slots[0], True, True, True)
            half(j + 1, slots[0], slots[1], True, True, True)
            half(j + 2, slots[1], slots[2], True, True, True)
            return carry

        lax.fori_loop(0, ntriples, triple, 0)
        half(nb - 4, slots[2], slots[0], True, True, True)
        half(nb - 3, slots[0], slots[1], False, True, True)
        half(nb - 2, slots[1], slots[2], False, True, True)
        gwait(rows2, sg2)
        iwait(di2, sd2)
        pltpu.sync_copy(rows2, acc.at[di2], add=True)
        scwait(rows0, di0, sc0)
        scwait(rows1, di1, sc1)

        if rem:
            roff = pl.multiple_of(base + nb * _K, 8)
            pltpu.sync_copy(ei.at[pl.ds(roff, rem)], si0.at[pl.ds(0, rem)])
            pltpu.sync_copy(ei.at[pl.ds(E + roff, rem)], dit)
            pltpu.sync_copy(hp.at[si0.at[pl.ds(0, rem)]],
                            rows0.at[pl.ds(0, rem)])
            pltpu.sync_copy(rows0.at[pl.ds(0, rem)], acc.at[dit], add=True)

        plsc.subcore_barrier()

        @pl.when(s < _NS - 1)
        def _():
            off = pl.multiple_of(s * rpt, 16)
            pltpu.sync_copy(acc.at[pl.ds(off, rpt)],
                            out.at[c, pl.ds(off, rpt)])

        @pl.when(s == _NS - 1)
        def _():
            off = (_NS - 1) * rpt
            pltpu.sync_copy(acc.at[pl.ds(off, rpt_last)],
                            out.at[c, pl.ds(off, rpt_last)])

    return scatter


def _matmul(x, W1):
    N, F = x.shape
    nbk = N // _R

    def body(x_ref, w_ref, o_ref):
        o_ref[...] = jnp.dot(x_ref[...], w_ref[...],
                             preferred_element_type=jnp.float32)

    return pl.pallas_call(
        body,
        grid=(nbk,),
        in_specs=[
            pl.BlockSpec((_R, F), lambda i: (i, 0)),
            pl.BlockSpec((F, F), lambda i: (0, 0)),
        ],
        out_specs=pl.BlockSpec((_R, F), lambda i: (i, 0)),
        out_shape=jax.ShapeDtypeStruct((N, F), jnp.float32),
    )(x, W1)


def _scale(xw, degp):
    N, F = xw.shape
    nbk = N // _R

    def body(xw_ref, d_ref, hp_ref, dinv_ref):
        deg = d_ref[0] + d_ref[1] + 1.0
        dv = lax.rsqrt(deg)
        dinv_ref[...] = dv
        hp_ref[...] = dv * xw_ref[...]

    return pl.pallas_call(
        body,
        grid=(nbk,),
        in_specs=[
            pl.BlockSpec((_R, F), lambda i: (i, 0)),
            pl.BlockSpec((2, _R, 1), lambda i: (0, i, 0)),
        ],
        out_specs=[
            pl.BlockSpec((_R, F), lambda i: (i, 0)),
            pl.BlockSpec((_R, 1), lambda i: (i, 0)),
        ],
        out_shape=[
            jax.ShapeDtypeStruct((N, F), jnp.float32),
            jax.ShapeDtypeStruct((N, 1), jnp.float32),
        ],
    )(xw, degp.reshape(_NC, N, 1))


def _layer_tail(sp, hp, dinv, b, g, be, res, Wn):
    N, F = hp.shape
    nbk = N // _R
    has_res = res is not None

    def body(*refs):
        if has_res:
            (sp_ref, hp_ref, dinv_ref, b_ref, g_ref, be_ref, res_ref, w_ref,
             z_ref, hpn_ref, y_s, ssum, ssq) = refs
        else:
            (sp_ref, hp_ref, dinv_ref, b_ref, g_ref, be_ref, w_ref,
             z_ref, hpn_ref, y_s, ssum, ssq) = refs
        p = pl.program_id(0)
        i = pl.program_id(1)

        @pl.when(p == 0)
        def _():
            y = ((sp_ref[0] + sp_ref[1] - hp_ref[...])
                 * dinv_ref[...] + b_ref[...])
            y_s[pl.ds(i * _R, _R), :] = y

            @pl.when(i == 0)
            def _():
                ssum[...] = jnp.zeros_like(ssum)
                ssq[...] = jnp.zeros_like(ssq)

            ssum[...] += jnp.sum(y, axis=0, keepdims=True)
            ssq[...] += jnp.sum(y * y, axis=0, keepdims=True)

        @pl.when(p == 1)
        def _():
            m = ssum[...] / N
            v = ssq[...] / N - m * m
            y = y_s[pl.ds(i * _R, _R), :]
            yn = (y - m) * lax.rsqrt(v + _EPS) * g_ref[...] + be_ref[...]
            z = _leaky(yn)
            if has_res:
                z = z + res_ref[...]
            z_ref[...] = z
            hpn_ref[...] = dinv_ref[...] * jnp.dot(
                z, w_ref[...], preferred_element_type=jnp.float32)

    frozen = lambda p, i: (i * (1 - p) + (nbk - 1) * p, 0)
    sp_spec = pl.BlockSpec((_NC, _R, F), lambda p, i: (0,) + frozen(p, i))
    in_specs = [
        sp_spec,
        pl.BlockSpec((_R, F), frozen),
        pl.BlockSpec((_R, 1), lambda p, i: (i, 0)),
        pl.BlockSpec((1, F), lambda p, i: (0, 0)),
        pl.BlockSpec((1, F), lambda p, i: (0, 0)),
        pl.BlockSpec((1, F), lambda p, i: (0, 0)),
    ]
    args = [sp, hp, dinv, b.reshape(1, F), g.reshape(1, F), be.reshape(1, F)]
    if has_res:
        in_specs.append(pl.BlockSpec((_R, F), lambda p, i: (i * p, 0)))
        args.append(res)
    in_specs.append(pl.BlockSpec((F, F), lambda p, i: (0, 0)))
    args.append(Wn)

    return pl.pallas_call(
        body,
        grid=(2, nbk),
        in_specs=in_specs,
        out_specs=[
            pl.BlockSpec((_R, F), lambda p, i: (i * p, 0)),
            pl.BlockSpec((_R, F), lambda p, i: (i * p, 0)),
        ],
        out_shape=[
            jax.ShapeDtypeStruct((N, F), jnp.float32),
            jax.ShapeDtypeStruct((N, F), jnp.float32),
        ],
        scratch_shapes=[
            pltpu.VMEM((N, F), jnp.float32),
            pltpu.VMEM((1, F), jnp.float32),
            pltpu.VMEM((1, F), jnp.float32),
        ],
    )(*args)


def _head(sp, hp, dinv, b, g, be, res, fcW1, fcb1, fcW2, fcb2):
    N, F = hp.shape
    D = fcW2.shape[1]
    nbk = N // _R

    def body(sp_ref, hp_ref, dinv_ref, b_ref, g_ref, be_ref, res_ref,
             w1_ref, b1_ref, w2_ref, b2_ref, out_ref, y_s, ssum, ssq, zsum):
        p = pl.program_id(0)
        i = pl.program_id(1)

        @pl.when(p == 0)
        def _():
            y = ((sp_ref[0] + sp_ref[1] - hp_ref[...])
                 * dinv_ref[...] + b_ref[...])
            y_s[pl.ds(i * _R, _R), :] = y

            @pl.when(i == 0)
            def _():
                ssum[...] = jnp.zeros_like(ssum)
                ssq[...] = jnp.zeros_like(ssq)

            ssum[...] += jnp.sum(y, axis=0, keepdims=True)
            ssq[...] += jnp.sum(y * y, axis=0, keepdims=True)

        @pl.when(p == 1)
        def _():
            m = ssum[...] / N
            v = ssq[...] / N - m * m
            y = y_s[pl.ds(i * _R, _R), :]
            yn = (y - m) * lax.rsqrt(v + _EPS) * g_ref[...] + be_ref[...]
            z = _leaky(yn) + res_ref[...]

            @pl.when(i == 0)
            def _():
                zsum[...] = jnp.zeros_like(zsum)

            zsum[...] += jnp.sum(z, axis=0, keepdims=True)

            @pl.when(i == nbk - 1)
            def _():
                pooled = zsum[...] / N
                h1 = _leaky(jnp.dot(pooled, w1_ref[...],
                                    preferred_element_type=jnp.float32)
                            + b1_ref[...])
                o = jnp.dot(h1, w2_ref[...],
                            preferred_element_type=jnp.float32) + b2_ref[...]
                nrm = jnp.sqrt(jnp.sum(o * o, axis=1, keepdims=True))
                out_ref[...] = o / jnp.maximum(nrm, 1e-12)

    frozen = lambda p, i: (i * (1 - p) + (nbk - 1) * p, 0)
    return pl.pallas_call(
        body,
        grid=(2, nbk),
        in_specs=[
            pl.BlockSpec((_NC, _R, F), lambda p, i: (0,) + frozen(p, i)),
            pl.BlockSpec((_R, F), frozen),
            pl.BlockSpec((_R, 1), lambda p, i: (i, 0)),
            pl.BlockSpec((1, F), lambda p, i: (0, 0)),
            pl.BlockSpec((1, F), lambda p, i: (0, 0)),
            pl.BlockSpec((1, F), lambda p, i: (0, 0)),
            pl.BlockSpec((_R, F), lambda p, i: (i * p, 0)),
            pl.BlockSpec((F, F), lambda p, i: (0, 0)),
            pl.BlockSpec((1, F), lambda p, i: (0, 0)),
            pl.BlockSpec((F, D), lambda p, i: (0, 0)),
            pl.BlockSpec((1, D), lambda p, i: (0, 0)),
        ],
        out_specs=pl.BlockSpec((1, D), lambda p, i: (0, 0)),
        out_shape=jax.ShapeDtypeStruct((1, D), jnp.float32),
        scratch_shapes=[
            pltpu.VMEM((N, F), jnp.float32),
            pltpu.VMEM((1, F), jnp.float32),
            pltpu.VMEM((1, F), jnp.float32),
            pltpu.VMEM((1, F), jnp.float32),
        ],
    )(sp, hp, dinv, b.reshape(1, F), g.reshape(1, F), be.reshape(1, F), res,
      fcW1, fcb1.reshape(1, F), fcW2, fcb2.reshape(1, D))


def kernel(x, edge_index, W1, b1, W2, b2, W3, b3, g1, be1, g2, be2, g3, be3,
           fcW1, fcb1, fcW2, fcb2):
    N, F = x.shape
    E = edge_index.shape[1]

    ei = edge_index.reshape(2 * E)
    degp = _deg_kernel(N, E)(ei, jnp.zeros((N,), jnp.float32)).reshape(_NC, N)
    xw = _matmul(x, W1)
    hp1, dinv = _scale(xw, degp)

    scat = _scatter_kernel(N, F, E)
    sp1 = scat(hp1, ei)
    z1, hp2 = _layer_tail(sp1, hp1, dinv, b1, g1, be1, None, W2)
    sp2 = scat(hp2, ei)
    z2, hp3 = _layer_tail(sp2, hp2, dinv, b2, g2, be2, z1, W3)
    sp3 = scat(hp3, ei)
    return _head(sp3, hp3, dinv, b3, g3, be3, z2, fcW1, fcb1, fcW2, fcb2)


# speedup vs baseline: 24.3220x; 1.0089x over previous
import functools

import jax
import jax.numpy as jnp
from jax import lax
from jax.experimental import pallas as pl
from jax.experimental.pallas import tpu as pltpu
from jax.experimental.pallas import tpu_sc as plsc

_NC = 2
_NS = 16
_K = 128
_R = 2000
_EPS = 1e-5


def _leaky(v):
    return jnp.where(v >= 0, v, 0.01 * v)


@functools.lru_cache(maxsize=None)
def _deg_kernel(N, E):
    NW = _NC * _NS
    epw = E // NW
    nb = epw // _K
    rem = epw - nb * _K
    assert E % NW == 0 and nb >= 10 and (nb - 2) % 4 == 0
    assert rem % 8 == 0
    nquads = (nb - 2) // 4 - 1
    mesh = plsc.VectorSubcoreMesh(core_axis_name="c", subcore_axis_name="s")

    @functools.partial(
        pl.kernel,
        mesh=mesh,
        out_type=jax.ShapeDtypeStruct((_NC * N,), jnp.float32),
        scratch_types=[
            pltpu.VMEM((_K,), jnp.int32),
            pltpu.VMEM((_K,), jnp.int32),
            pltpu.VMEM((_K,), jnp.int32),
            pltpu.VMEM((_K,), jnp.int32),
            pltpu.VMEM((max(rem, 8),), jnp.int32),
            pltpu.VMEM((_K,), jnp.float32),
            pltpu.VMEM((N,), jnp.float32),
            pltpu.VMEM_SHARED((N,), jnp.float32),
            pltpu.SemaphoreType.DMA,
            pltpu.SemaphoreType.DMA,
            pltpu.SemaphoreType.DMA,
            pltpu.SemaphoreType.DMA,
            pltpu.SemaphoreType.DMA,
            pltpu.SemaphoreType.DMA,
            pltpu.SemaphoreType.DMA,
            pltpu.SemaphoreType.DMA,
        ],
    )
    def deg(ei, zeros, out, idx0, idx1, idx2, idx3, idxt, ones, stage, acc,
            si0, si1, si2, si3, sc0, sc1, sc2, sc3):
        c = lax.axis_index("c")
        s = lax.axis_index("s")
        base = E + (c * _NS + s) * epw

        @pl.when(s == 0)
        def _():
            pltpu.sync_copy(zeros, stage)
            pltpu.sync_copy(stage, acc)

        for j in range(_K // 16):
            ones[pl.ds(16 * j, 16)] = jnp.ones((16,), jnp.float32)
        plsc.subcore_barrier()

        slots = ((idx0, si0, sc0), (idx1, si1, sc1),
                 (idx2, si2, sc2), (idx3, si3, sc3))

        def load(i, slot):
            off = pl.multiple_of(base + i * _K, 8)
            pltpu.async_copy(ei.at[pl.ds(off, _K)], slot[0], slot[1])

        def scwait(slot):
            pltpu.make_async_copy(ones, acc.at[slot[0]], slot[2]).wait()

        def body(j, A, W, ld, do_scwait):
            pltpu.make_async_copy(ei.at[pl.ds(0, _K)], A[0], A[1]).wait()
            if do_scwait:
                scwait(W)
            pltpu.async_copy(ones, acc.at[A[0]], A[2], add=True)
            if ld:
                load(j + 2, W)

        load(0, slots[0])
        load(1, slots[1])
        body(0, slots[0], slots[2], True, False)
        body(1, slots[1], slots[3], True, False)

        def quad(t, carry):
            j = 4 * t + 2
            body(j, slots[2], slots[0], True, True)
            body(j + 1, slots[3], slots[1], True, True)
            body(j + 2, slots[0], slots[2], True, True)
            body(j + 3, slots[1], slots[3], True, True)
            return carry

        lax.fori_loop(0, nquads, quad, 0)
        j0 = nb - 4
        body(j0, slots[2], slots[0], True, True)
        body(j0 + 1, slots[3], slots[1], True, True)
        body(j0 + 2, slots[0], slots[2], False, True)
        body(j0 + 3, slots[1], slots[3], False, True)
        scwait(slots[0])
        scwait(slots[1])
        if rem:
            roff = pl.multiple_of(base + nb * _K, 8)
            pltpu.sync_copy(ei.at[pl.ds(roff, rem)], idxt)
            pltpu.sync_copy(ones.at[pl.ds(0, rem)], acc.at[idxt], add=True)

        plsc.subcore_barrier()

        @pl.when(s == 0)
        def _():
            coff = pl.multiple_of(c * N, 8)
            pltpu.sync_copy(acc, stage)
            pltpu.sync_copy(stage, out.at[pl.ds(coff, N)])

    return deg


@functools.lru_cache(maxsize=None)
def _scatter_kernel(N, F, E):
    NW = _NC * _NS
    epw = E // NW
    nb = epw // _K
    rem = epw - nb * _K
    assert E % NW == 0 and nb >= 9 and (nb - 3) % 3 == 0
    assert rem % 8 == 0
    rpt = (-(-N // _NS) + 15) // 16 * 16
    rpt_last = N - (_NS - 1) * rpt
    assert rpt_last > 0
    ntriples = (nb - 3) // 3 - 1
    mesh = plsc.VectorSubcoreMesh(core_axis_name="c", subcore_axis_name="s")

    @functools.partial(
        pl.kernel,
        mesh=mesh,
        out_type=jax.ShapeDtypeStruct((_NC, N, F), jnp.float32),
        scratch_types=[
            pltpu.VMEM((_K,), jnp.int32),
            pltpu.VMEM((_K,), jnp.int32),
            pltpu.VMEM((_K,), jnp.int32),
            pltpu.VMEM((_K,), jnp.int32),
            pltpu.VMEM((_K,), jnp.int32),
            pltpu.VMEM((_K,), jnp.int32),
            pltpu.VMEM((max(rem, 8),), jnp.int32),
            pltpu.VMEM((_K, F), jnp.float32),
            pltpu.VMEM((_K, F), jnp.float32),
            pltpu.VMEM((_K, F), jnp.float32),
            pltpu.VMEM_SHARED((N, F), jnp.float32),
            pltpu.SemaphoreType.DMA,
            pltpu.SemaphoreType.DMA,
            pltpu.SemaphoreType.DMA,
            pltpu.SemaphoreType.DMA,
            pltpu.SemaphoreType.DMA,
            pltpu.SemaphoreType.DMA,
            pltpu.SemaphoreType.DMA,
            pltpu.SemaphoreType.DMA,
            pltpu.SemaphoreType.DMA,
            pltpu.SemaphoreType.DMA,
            pltpu.SemaphoreType.DMA,
            pltpu.SemaphoreType.DMA,
        ],
    )
    def scatter(hp, ei, out, si0, si1, si2, di0, di1, di2, dit,
                rows0, rows1, rows2, acc,
                ss0, ss1, ss2, sd0, sd1, sd2, sg0, sg1, sg2, sc0, sc1, sc2):
        c = lax.axis_index("c")
        s = lax.axis_index("s")
        base = (c * _NS + s) * epw

        @pl.when(s < _NS - 1)
        def _():
            off = pl.multiple_of(s * rpt, 16)
            pltpu.sync_copy(hp.at[pl.ds(off, rpt)], acc.at[pl.ds(off, rpt)])

        @pl.when(s == _NS - 1)
        def _():
            off = (_NS - 1) * rpt
            pltpu.sync_copy(hp.at[pl.ds(off, rpt_last)],
                            acc.at[pl.ds(off, rpt_last)])

        plsc.subcore_barrier()

        def srcload(i, ref, sem):
            off = pl.multiple_of(base + i * _K, 8)
            pltpu.async_copy(ei.at[pl.ds(off, _K)], ref, sem)

        def dstload(i, ref, sem):
            off = pl.multiple_of(E + base + i * _K, 8)
            pltpu.async_copy(ei.at[pl.ds(off, _K)], ref, sem)

        def iwait(ref, sem):
            pltpu.make_async_copy(ei.at[pl.ds(0, _K)], ref, sem).wait()

        def gwait(rref, sem):
            pltpu.make_async_copy(hp.at[pl.ds(0, _K)], rref, sem).wait()

        def scwait(rref, iref, sem):
            pltpu.make_async_copy(rref, acc.at[iref], sem).wait()

        slots = ((si0, di0, rows0, ss0, sd0, sg0, sc0),
                 (si1, di1, rows1, ss1, sd1, sg1, sc1),
                 (si2, di2, rows2, ss2, sd2, sg2, sc2))

        def half(j, A, B, ld_src, ld_dst, do_scwait):
            siA, diA, rA, ssA, sdA, sgA, scA = A
            siB, diB, rB, ssB, sdB, sgB, scB = B
            gwait(rA, sgA)
            iwait(siB, ssB)
            if do_scwait:
                scwait(rB, diB, scB)
            pltpu.async_copy(hp.at[siB], rB, sgB)
            if ld_src:
                srcload(j + 3, siA, ssA)
            if ld_dst:
                dstload(j + 1, diB, sdB)
            iwait(diA, sdA)
            pltpu.async_copy(rA, acc.at[diA], scA, add=True)

        srcload(0, si0, ss0)
        dstload(0, di0, sd0)
        iwait(si0, ss0)
        pltpu.async_copy(hp.at[si0], rows0, sg0)
        srcload(1, si1, ss1)
        srcload(2, si2, ss2)

        half(0, slots[0], slots[1], True, True, False)
        half(1, slots[1], slots[2], True, True, False)

        def triple(t, carry):
            j = 3 * t + 2
            half(j, slots[2], slots[0], True, True, True)
            half(j + 1, slots[0], slots[1], True, True, True)
            half(j + 2, slots[1], slots[2], True, True, True)
            return carry

        lax.fori_loop(0, ntriples, triple, 0)
        half(nb - 4, slots[2], slots[0], True, True, True)
        half(nb - 3, slots[0], slots[1], False, True, True)
        half(nb - 2, slots[1], slots[2], False, True, True)
        gwait(rows2, sg2)
        iwait(di2, sd2)
        pltpu.sync_copy(rows2, acc.at[di2], add=True)
        scwait(rows0, di0, sc0)
        scwait(rows1, di1, sc1)

        if rem:
            roff = pl.multiple_of(base + nb * _K, 8)
            pltpu.sync_copy(ei.at[pl.ds(roff, rem)], si0.at[pl.ds(0, rem)])
            pltpu.sync_copy(ei.at[pl.ds(E + roff, rem)], dit)
            pltpu.sync_copy(hp.at[si0.at[pl.ds(0, rem)]],
                            rows0.at[pl.ds(0, rem)])
            pltpu.sync_copy(rows0.at[pl.ds(0, rem)], acc.at[dit], add=True)

        plsc.subcore_barrier()

        @pl.when(s < _NS - 1)
        def _():
            off = pl.multiple_of(s * rpt, 16)
            pltpu.sync_copy(acc.at[pl.ds(off, rpt)],
                            out.at[c, pl.ds(off, rpt)])

        @pl.when(s == _NS - 1)
        def _():
            off = (_NS - 1) * rpt
            pltpu.sync_copy(acc.at[pl.ds(off, rpt_last)],
                            out.at[c, pl.ds(off, rpt_last)])

    return scatter


def _matmul(x, W1):
    N, F = x.shape
    nbk = N // _R

    def body(x_ref, w_ref, o_ref):
        o_ref[...] = jnp.dot(x_ref[...], w_ref[...],
                             preferred_element_type=jnp.float32)

    return pl.pallas_call(
        body,
        grid=(nbk,),
        in_specs=[
            pl.BlockSpec((_R, F), lambda i: (i, 0)),
            pl.BlockSpec((F, F), lambda i: (0, 0)),
        ],
        out_specs=pl.BlockSpec((_R, F), lambda i: (i, 0)),
        out_shape=jax.ShapeDtypeStruct((N, F), jnp.float32),
    )(x, W1)


def _scale(xw, degp):
    N, F = xw.shape
    nbk = N // _R

    def body(xw_ref, d_ref, hp_ref, dinv_ref):
        deg = d_ref[0] + d_ref[1] + 1.0
        dv = lax.rsqrt(deg)
        dinv_ref[...] = dv
        hp_ref[...] = dv * xw_ref[...]

    return pl.pallas_call(
        body,
        grid=(nbk,),
        in_specs=[
            pl.BlockSpec((_R, F), lambda i: (i, 0)),
            pl.BlockSpec((2, _R, 1), lambda i: (0, i, 0)),
        ],
        out_specs=[
            pl.BlockSpec((_R, F), lambda i: (i, 0)),
            pl.BlockSpec((_R, 1), lambda i: (i, 0)),
        ],
        out_shape=[
            jax.ShapeDtypeStruct((N, F), jnp.float32),
            jax.ShapeDtypeStruct((N, 1), jnp.float32),
        ],
    )(xw, degp.reshape(_NC, N, 1))


def _layer_tail(sp, hp, dinv, b, g, be, res, Wn):
    N, F = hp.shape
    nbk = N // _R
    has_res = res is not None

    def body(*refs):
        if has_res:
            (sp_ref, hp_ref, dinv_ref, b_ref, g_ref, be_ref, res_ref, w_ref,
             z_ref, hpn_ref, y_s, ssum, ssq) = refs
        else:
            (sp_ref, hp_ref, dinv_ref, b_ref, g_ref, be_ref, w_ref,
             z_ref, hpn_ref, y_s, ssum, ssq) = refs
        p = pl.program_id(0)
        i = pl.program_id(1)

        @pl.when(p == 0)
        def _():
            y = ((sp_ref[0] + sp_ref[1] - hp_ref[...])
                 * dinv_ref[...] + b_ref[...])
            y_s[pl.ds(i * _R, _R), :] = y

            @pl.when(i == 0)
            def _():
                ssum[...] = jnp.zeros_like(ssum)
                ssq[...] = jnp.zeros_like(ssq)

            ssum[...] += jnp.sum(y, axis=0, keepdims=True)
            ssq[...] += jnp.sum(y * y, axis=0, keepdims=True)

        @pl.when(p == 1)
        def _():
            m = ssum[...] / N
            v = ssq[...] / N - m * m
            y = y_s[pl.ds(i * _R, _R), :]
            yn = (y - m) * lax.rsqrt(v + _EPS) * g_ref[...] + be_ref[...]
            z = _leaky(yn)
            if has_res:
                z = z + res_ref[...]
            z_ref[...] = z
            hpn_ref[...] = dinv_ref[...] * jnp.dot(
                z, w_ref[...], preferred_element_type=jnp.float32)

    frozen = lambda p, i: (i * (1 - p) + (nbk - 1) * p, 0)
    sp_spec = pl.BlockSpec((_NC, _R, F), lambda p, i: (0,) + frozen(p, i))
    in_specs = [
        sp_spec,
        pl.BlockSpec((_R, F), frozen),
        pl.BlockSpec((_R, 1), lambda p, i: (i, 0)),
        pl.BlockSpec((1, F), lambda p, i: (0, 0)),
        pl.BlockSpec((1, F), lambda p, i: (0, 0)),
        pl.BlockSpec((1, F), lambda p, i: (0, 0)),
    ]
    args = [sp, hp, dinv, b.reshape(1, F), g.reshape(1, F), be.reshape(1, F)]
    if has_res:
        in_specs.append(pl.BlockSpec((_R, F), lambda p, i: (i * p, 0)))
        args.append(res)
    in_specs.append(pl.BlockSpec((F, F), lambda p, i: (0, 0)))
    args.append(Wn)

    return pl.pallas_call(
        body,
        grid=(2, nbk),
        in_specs=in_specs,
        out_specs=[
            pl.BlockSpec((_R, F), lambda p, i: (i * p, 0)),
            pl.BlockSpec((_R, F), lambda p, i: (i * p, 0)),
        ],
        out_shape=[
            jax.ShapeDtypeStruct((N, F), jnp.float32),
            jax.ShapeDtypeStruct((N, F), jnp.float32),
        ],
        scratch_shapes=[
            pltpu.VMEM((N, F), jnp.float32),
            pltpu.VMEM((1, F), jnp.float32),
            pltpu.VMEM((1, F), jnp.float32),
        ],
    )(*args)


def _head(sp, hp, dinv, b, g, be, res, fcW1, fcb1, fcW2, fcb2):
    N, F = hp.shape
    D = fcW2.shape[1]
    nbk = N // _R

    def body(sp_ref, hp_ref, dinv_ref, b_ref, g_ref, be_ref, res_ref,
             w1_ref, b1_ref, w2_ref, b2_ref, out_ref, y_s, ssum, ssq, zsum):
        p = pl.program_id(0)
        i = pl.program_id(1)

        @pl.when(p == 0)
        def _():
            y = ((sp_ref[0] + sp_ref[1] - hp_ref[...])
                 * dinv_ref[...] + b_ref[...])
            y_s[pl.ds(i * _R, _R), :] = y

            @pl.when(i == 0)
            def _():
                ssum[...] = jnp.zeros_like(ssum)
                ssq[...] = jnp.zeros_like(ssq)

            ssum[...] += jnp.sum(y, axis=0, keepdims=True)
            ssq[...] += jnp.sum(y * y, axis=0, keepdims=True)

        @pl.when(p == 1)
        def _():
            m = ssum[...] / N
            v = ssq[...] / N - m * m
            y = y_s[pl.ds(i * _R, _R), :]
            yn = (y - m) * lax.rsqrt(v + _EPS) * g_ref[...] + be_ref[...]
            z = _leaky(yn) + res_ref[...]

            @pl.when(i == 0)
            def _():
                zsum[...] = jnp.zeros_like(zsum)

            zsum[...] += jnp.sum(z, axis=0, keepdims=True)

            @pl.when(i == nbk - 1)
            def _():
                pooled = zsum[...] / N
                h1 = _leaky(jnp.dot(pooled, w1_ref[...],
                                    preferred_element_type=jnp.float32)
                            + b1_ref[...])
                o = jnp.dot(h1, w2_ref[...],
                            preferred_element_type=jnp.float32) + b2_ref[...]
                nrm = jnp.sqrt(jnp.sum(o * o, axis=1, keepdims=True))
                out_ref[...] = o / jnp.maximum(nrm, 1e-12)

    frozen = lambda p, i: (i * (1 - p) + (nbk - 1) * p, 0)
    return pl.pallas_call(
        body,
        grid=(2, nbk),
        in_specs=[
            pl.BlockSpec((_NC, _R, F), lambda p, i: (0,) + frozen(p, i)),
            pl.BlockSpec((_R, F), frozen),
            pl.BlockSpec((_R, 1), lambda p, i: (i, 0)),
            pl.BlockSpec((1, F), lambda p, i: (0, 0)),
            pl.BlockSpec((1, F), lambda p, i: (0, 0)),
            pl.BlockSpec((1, F), lambda p, i: (0, 0)),
            pl.BlockSpec((_R, F), lambda p, i: (i * p, 0)),
            pl.BlockSpec((F, F), lambda p, i: (0, 0)),
            pl.BlockSpec((1, F), lambda p, i: (0, 0)),
            pl.BlockSpec((F, D), lambda p, i: (0, 0)),
            pl.BlockSpec((1, D), lambda p, i: (0, 0)),
        ],
        out_specs=pl.BlockSpec((1, D), lambda p, i: (0, 0)),
        out_shape=jax.ShapeDtypeStruct((1, D), jnp.float32),
        scratch_shapes=[
            pltpu.VMEM((N, F), jnp.float32),
            pltpu.VMEM((1, F), jnp.float32),
            pltpu.VMEM((1, F), jnp.float32),
            pltpu.VMEM((1, F), jnp.float32),
        ],
    )(sp, hp, dinv, b.reshape(1, F), g.reshape(1, F), be.reshape(1, F), res,
      fcW1, fcb1.reshape(1, F), fcW2, fcb2.reshape(1, D))


def kernel(x, edge_index, W1, b1, W2, b2, W3, b3, g1, be1, g2, be2, g3, be3,
           fcW1, fcb1, fcW2, fcb2):
    N, F = x.shape
    E = edge_index.shape[1]

    ei = edge_index.reshape(2 * E)
    degp = _deg_kernel(N, E)(ei, jnp.zeros((N,), jnp.float32)).reshape(_NC, N)
    xw = _matmul(x, W1)
    hp1, dinv = _scale(xw, degp)

    scat = _scatter_kernel(N, F, E)
    sp1 = scat(hp1, ei)
    z1, hp2 = _layer_tail(sp1, hp1, dinv, b1, g1, be1, None, W2)
    sp2 = scat(hp2, ei)
    z2, hp3 = _layer_tail(sp2, hp2, dinv, b2, g2, be2, z1, W3)
    sp3 = scat(hp3, ei)
    return _head(sp3, hp3, dinv, b3, g3, be3, z2, fcW1, fcb1, fcW2, fcb2)


# speedup vs baseline: 24.5892x; 1.0110x over previous
import functools

import jax
import jax.numpy as jnp
from jax import lax
from jax.experimental import pallas as pl
from jax.experimental.pallas import tpu as pltpu
from jax.experimental.pallas import tpu_sc as plsc

_NC = 2
_NS = 16
_K = 128
_R = 5000
_EPS = 1e-5


def _leaky(v):
    return jnp.where(v >= 0, v, 0.01 * v)


@functools.lru_cache(maxsize=None)
def _deg_kernel(N, E):
    NW = _NC * _NS
    epw = E // NW
    nb = epw // _K
    rem = epw - nb * _K
    assert E % NW == 0 and nb >= 10 and (nb - 2) % 4 == 0
    assert rem % 8 == 0
    nquads = (nb - 2) // 4 - 1
    mesh = plsc.VectorSubcoreMesh(core_axis_name="c", subcore_axis_name="s")

    @functools.partial(
        pl.kernel,
        mesh=mesh,
        out_type=jax.ShapeDtypeStruct((_NC * N,), jnp.float32),
        scratch_types=[
            pltpu.VMEM((_K,), jnp.int32),
            pltpu.VMEM((_K,), jnp.int32),
            pltpu.VMEM((_K,), jnp.int32),
            pltpu.VMEM((_K,), jnp.int32),
            pltpu.VMEM((max(rem, 8),), jnp.int32),
            pltpu.VMEM((_K,), jnp.float32),
            pltpu.VMEM((N,), jnp.float32),
            pltpu.VMEM_SHARED((N,), jnp.float32),
            pltpu.SemaphoreType.DMA,
            pltpu.SemaphoreType.DMA,
            pltpu.SemaphoreType.DMA,
            pltpu.SemaphoreType.DMA,
            pltpu.SemaphoreType.DMA,
            pltpu.SemaphoreType.DMA,
            pltpu.SemaphoreType.DMA,
            pltpu.SemaphoreType.DMA,
        ],
    )
    def deg(ei, zeros, out, idx0, idx1, idx2, idx3, idxt, ones, stage, acc,
            si0, si1, si2, si3, sc0, sc1, sc2, sc3):
        c = lax.axis_index("c")
        s = lax.axis_index("s")
        base = E + (c * _NS + s) * epw

        @pl.when(s == 0)
        def _():
            pltpu.sync_copy(zeros, stage)
            pltpu.sync_copy(stage, acc)

        for j in range(_K // 16):
            ones[pl.ds(16 * j, 16)] = jnp.ones((16,), jnp.float32)
        plsc.subcore_barrier()

        slots = ((idx0, si0, sc0), (idx1, si1, sc1),
                 (idx2, si2, sc2), (idx3, si3, sc3))

        def load(i, slot):
            off = pl.multiple_of(base + i * _K, 8)
            pltpu.async_copy(ei.at[pl.ds(off, _K)], slot[0], slot[1])

        def scwait(slot):
            pltpu.make_async_copy(ones, acc.at[slot[0]], slot[2]).wait()

        def body(j, A, W, ld, do_scwait):
            pltpu.make_async_copy(ei.at[pl.ds(0, _K)], A[0], A[1]).wait()
            if do_scwait:
                scwait(W)
            pltpu.async_copy(ones, acc.at[A[0]], A[2], add=True)
            if ld:
                load(j + 2, W)

        load(0, slots[0])
        load(1, slots[1])
        body(0, slots[0], slots[2], True, False)
        body(1, slots[1], slots[3], True, False)

        def quad(t, carry):
            j = 4 * t + 2
            body(j, slots[2], slots[0], True, True)
            body(j + 1, slots[3], slots[1], True, True)
            body(j + 2, slots[0], slots[2], True, True)
            body(j + 3, slots[1], slots[3], True, True)
            return carry

        lax.fori_loop(0, nquads, quad, 0)
        j0 = nb - 4
        body(j0, slots[2], slots[0], True, True)
        body(j0 + 1, slots[3], slots[1], True, True)
        body(j0 + 2, slots[0], slots[2], False, True)
        body(j0 + 3, slots[1], slots[3], False, True)
        scwait(slots[0])
        scwait(slots[1])
        if rem:
            roff = pl.multiple_of(base + nb * _K, 8)
            pltpu.sync_copy(ei.at[pl.ds(roff, rem)], idxt)
            pltpu.sync_copy(ones.at[pl.ds(0, rem)], acc.at[idxt], add=True)

        plsc.subcore_barrier()

        @pl.when(s == 0)
        def _():
            coff = pl.multiple_of(c * N, 8)
            pltpu.sync_copy(acc, stage)
            pltpu.sync_copy(stage, out.at[pl.ds(coff, N)])

    return deg


@functools.lru_cache(maxsize=None)
def _scatter_kernel(N, F, E):
    NW = _NC * _NS
    epw = E // NW
    nb = epw // _K
    rem = epw - nb * _K
    assert E % NW == 0 and nb >= 9 and (nb - 3) % 3 == 0
    assert rem % 8 == 0
    rpt = (-(-N // _NS) + 15) // 16 * 16
    rpt_last = N - (_NS - 1) * rpt
    assert rpt_last > 0
    ntriples = (nb - 3) // 3 - 1
    mesh = plsc.VectorSubcoreMesh(core_axis_name="c", subcore_axis_name="s")

    @functools.partial(
        pl.kernel,
        mesh=mesh,
        out_type=jax.ShapeDtypeStruct((_NC, N, F), jnp.float32),
        scratch_types=[
            pltpu.VMEM((_K,), jnp.int32),
            pltpu.VMEM((_K,), jnp.int32),
            pltpu.VMEM((_K,), jnp.int32),
            pltpu.VMEM((_K,), jnp.int32),
            pltpu.VMEM((_K,), jnp.int32),
            pltpu.VMEM((_K,), jnp.int32),
            pltpu.VMEM((max(rem, 8),), jnp.int32),
            pltpu.VMEM((_K, F), jnp.float32),
            pltpu.VMEM((_K, F), jnp.float32),
            pltpu.VMEM((_K, F), jnp.float32),
            pltpu.VMEM_SHARED((N, F), jnp.float32),
            pltpu.SemaphoreType.DMA,
            pltpu.SemaphoreType.DMA,
            pltpu.SemaphoreType.DMA,
            pltpu.SemaphoreType.DMA,
            pltpu.SemaphoreType.DMA,
            pltpu.SemaphoreType.DMA,
            pltpu.SemaphoreType.DMA,
            pltpu.SemaphoreType.DMA,
            pltpu.SemaphoreType.DMA,
            pltpu.SemaphoreType.DMA,
            pltpu.SemaphoreType.DMA,
            pltpu.SemaphoreType.DMA,
        ],
    )
    def scatter(hp, ei, out, si0, si1, si2, di0, di1, di2, dit,
                rows0, rows1, rows2, acc,
                ss0, ss1, ss2, sd0, sd1, sd2, sg0, sg1, sg2, sc0, sc1, sc2):
        c = lax.axis_index("c")
        s = lax.axis_index("s")
        base = (c * _NS + s) * epw

        @pl.when(s < _NS - 1)
        def _():
            off = pl.multiple_of(s * rpt, 16)
            pltpu.sync_copy(hp.at[pl.ds(off, rpt)], acc.at[pl.ds(off, rpt)])

        @pl.when(s == _NS - 1)
        def _():
            off = (_NS - 1) * rpt
            pltpu.sync_copy(hp.at[pl.ds(off, rpt_last)],
                            acc.at[pl.ds(off, rpt_last)])

        plsc.subcore_barrier()

        def srcload(i, ref, sem):
            off = pl.multiple_of(base + i * _K, 8)
            pltpu.async_copy(ei.at[pl.ds(off, _K)], ref, sem)

        def dstload(i, ref, sem):
            off = pl.multiple_of(E + base + i * _K, 8)
            pltpu.async_copy(ei.at[pl.ds(off, _K)], ref, sem)

        def iwait(ref, sem):
            pltpu.make_async_copy(ei.at[pl.ds(0, _K)], ref, sem).wait()

        def gwait(rref, sem):
            pltpu.make_async_copy(hp.at[pl.ds(0, _K)], rref, sem).wait()

        def scwait(rref, iref, sem):
            pltpu.make_async_copy(rref, acc.at[iref], sem).wait()

        slots = ((si0, di0, rows0, ss0, sd0, sg0, sc0),
                 (si1, di1, rows1, ss1, sd1, sg1, sc1),
                 (si2, di2, rows2, ss2, sd2, sg2, sc2))

        def half(j, A, B, ld_src, ld_dst, do_scwait):
            siA, diA, rA, ssA, sdA, sgA, scA = A
            siB, diB, rB, ssB, sdB, sgB, scB = B
            gwait(rA, sgA)
            iwait(siB, ssB)
            if do_scwait:
                scwait(rB, diB, scB)
            pltpu.async_copy(hp.at[siB], rB, sgB)
            if ld_src:
                srcload(j + 3, siA, ssA)
            if ld_dst:
                dstload(j + 1, diB, sdB)
            iwait(diA, sdA)
            pltpu.async_copy(rA, acc.at[diA], scA, add=True)

        srcload(0, si0, ss0)
        dstload(0, di0, sd0)
        iwait(si0, ss0)
        pltpu.async_copy(hp.at[si0], rows0, sg0)
        srcload(1, si1, ss1)
        srcload(2, si2, ss2)

        half(0, slots[0], slots[1], True, True, False)
        half(1, slots[1], slots[2], True, True, False)

        def triple(t, carry):
            j = 3 * t + 2
            half(j, slots[2], slots[0], True, True, True)
            half(j + 1, slots[0], slots[1], True, True, True)
            half(j + 2, slots[1], slots[2], True, True, True)
            return carry

        lax.fori_loop(0, ntriples, triple, 0)
        half(nb - 4, slots[2], slots[0], True, True, True)
        half(nb - 3, slots[0], slots[1], False, True, True)
        half(nb - 2, slots[1], slots[2], False, True, True)
        gwait(rows2, sg2)
        iwait(di2, sd2)
        pltpu.sync_copy(rows2, acc.at[di2], add=True)
        scwait(rows0, di0, sc0)
        scwait(rows1, di1, sc1)

        if rem:
            roff = pl.multiple_of(base + nb * _K, 8)
            pltpu.sync_copy(ei.at[pl.ds(roff, rem)], si0.at[pl.ds(0, rem)])
            pltpu.sync_copy(ei.at[pl.ds(E + roff, rem)], dit)
            pltpu.sync_copy(hp.at[si0.at[pl.ds(0, rem)]],
                            rows0.at[pl.ds(0, rem)])
            pltpu.sync_copy(rows0.at[pl.ds(0, rem)], acc.at[dit], add=True)

        plsc.subcore_barrier()

        @pl.when(s < _NS - 1)
        def _():
            off = pl.multiple_of(s * rpt, 16)
            pltpu.sync_copy(acc.at[pl.ds(off, rpt)],
                            out.at[c, pl.ds(off, rpt)])

        @pl.when(s == _NS - 1)
        def _():
            off = (_NS - 1) * rpt
            pltpu.sync_copy(acc.at[pl.ds(off, rpt_last)],
                            out.at[c, pl.ds(off, rpt_last)])

    return scatter


def _matmul(x, W1):
    N, F = x.shape
    nbk = N // _R

    def body(x_ref, w_ref, o_ref):
        o_ref[...] = jnp.dot(x_ref[...], w_ref[...],
                             preferred_element_type=jnp.float32)

    return pl.pallas_call(
        body,
        grid=(nbk,),
        in_specs=[
            pl.BlockSpec((_R, F), lambda i: (i, 0)),
            pl.BlockSpec((F, F), lambda i: (0, 0)),
        ],
        out_specs=pl.BlockSpec((_R, F), lambda i: (i, 0)),
        out_shape=jax.ShapeDtypeStruct((N, F), jnp.float32),
    )(x, W1)


def _scale(xw, degp):
    N, F = xw.shape
    nbk = N // _R

    def body(xw_ref, d_ref, hp_ref, dinv_ref):
        deg = d_ref[0] + d_ref[1] + 1.0
        dv = lax.rsqrt(deg)
        dinv_ref[...] = dv
        hp_ref[...] = dv * xw_ref[...]

    return pl.pallas_call(
        body,
        grid=(nbk,),
        in_specs=[
            pl.BlockSpec((_R, F), lambda i: (i, 0)),
            pl.BlockSpec((2, _R, 1), lambda i: (0, i, 0)),
        ],
        out_specs=[
            pl.BlockSpec((_R, F), lambda i: (i, 0)),
            pl.BlockSpec((_R, 1), lambda i: (i, 0)),
        ],
        out_shape=[
            jax.ShapeDtypeStruct((N, F), jnp.float32),
            jax.ShapeDtypeStruct((N, 1), jnp.float32),
        ],
    )(xw, degp.reshape(_NC, N, 1))


def _layer_tail(sp, hp, dinv, b, g, be, res, Wn):
    N, F = hp.shape
    nbk = N // _R
    has_res = res is not None

    def body(*refs):
        if has_res:
            (sp_ref, hp_ref, dinv_ref, b_ref, g_ref, be_ref, res_ref, w_ref,
             z_ref, hpn_ref, y_s, ssum, ssq) = refs
        else:
            (sp_ref, hp_ref, dinv_ref, b_ref, g_ref, be_ref, w_ref,
             z_ref, hpn_ref, y_s, ssum, ssq) = refs
        p = pl.program_id(0)
        i = pl.program_id(1)

        @pl.when(p == 0)
        def _():
            y = ((sp_ref[0] + sp_ref[1] - hp_ref[...])
                 * dinv_ref[...] + b_ref[...])
            y_s[pl.ds(i * _R, _R), :] = y

            @pl.when(i == 0)
            def _():
                ssum[...] = jnp.zeros_like(ssum)
                ssq[...] = jnp.zeros_like(ssq)

            ssum[...] += jnp.sum(y, axis=0, keepdims=True)
            ssq[...] += jnp.sum(y * y, axis=0, keepdims=True)

        @pl.when(p == 1)
        def _():
            m = ssum[...] / N
            v = ssq[...] / N - m * m
            y = y_s[pl.ds(i * _R, _R), :]
            yn = (y - m) * lax.rsqrt(v + _EPS) * g_ref[...] + be_ref[...]
            z = _leaky(yn)
            if has_res:
                z = z + res_ref[...]
            z_ref[...] = z
            hpn_ref[...] = dinv_ref[...] * jnp.dot(
                z, w_ref[...], preferred_element_type=jnp.float32)

    frozen = lambda p, i: (i * (1 - p) + (nbk - 1) * p, 0)
    sp_spec = pl.BlockSpec((_NC, _R, F), lambda p, i: (0,) + frozen(p, i))
    in_specs = [
        sp_spec,
        pl.BlockSpec((_R, F), frozen),
        pl.BlockSpec((_R, 1), lambda p, i: (i, 0)),
        pl.BlockSpec((1, F), lambda p, i: (0, 0)),
        pl.BlockSpec((1, F), lambda p, i: (0, 0)),
        pl.BlockSpec((1, F), lambda p, i: (0, 0)),
    ]
    args = [sp, hp, dinv, b.reshape(1, F), g.reshape(1, F), be.reshape(1, F)]
    if has_res:
        in_specs.append(pl.BlockSpec((_R, F), lambda p, i: (i * p, 0)))
        args.append(res)
    in_specs.append(pl.BlockSpec((F, F), lambda p, i: (0, 0)))
    args.append(Wn)

    return pl.pallas_call(
        body,
        grid=(2, nbk),
        in_specs=in_specs,
        out_specs=[
            pl.BlockSpec((_R, F), lambda p, i: (i * p, 0)),
            pl.BlockSpec((_R, F), lambda p, i: (i * p, 0)),
        ],
        out_shape=[
            jax.ShapeDtypeStruct((N, F), jnp.float32),
            jax.ShapeDtypeStruct((N, F), jnp.float32),
        ],
        scratch_shapes=[
            pltpu.VMEM((N, F), jnp.float32),
            pltpu.VMEM((1, F), jnp.float32),
            pltpu.VMEM((1, F), jnp.float32),
        ],
    )(*args)


def _head(sp, hp, dinv, b, g, be, res, fcW1, fcb1, fcW2, fcb2):
    N, F = hp.shape
    D = fcW2.shape[1]
    nbk = N // _R

    def body(sp_ref, hp_ref, dinv_ref, b_ref, g_ref, be_ref, res_ref,
             w1_ref, b1_ref, w2_ref, b2_ref, out_ref, y_s, ssum, ssq, zsum):
        p = pl.program_id(0)
        i = pl.program_id(1)

        @pl.when(p == 0)
        def _():
            y = ((sp_ref[0] + sp_ref[1] - hp_ref[...])
                 * dinv_ref[...] + b_ref[...])
            y_s[pl.ds(i * _R, _R), :] = y

            @pl.when(i == 0)
            def _():
                ssum[...] = jnp.zeros_like(ssum)
                ssq[...] = jnp.zeros_like(ssq)

            ssum[...] += jnp.sum(y, axis=0, keepdims=True)
            ssq[...] += jnp.sum(y * y, axis=0, keepdims=True)

        @pl.when(p == 1)
        def _():
            m = ssum[...] / N
            v = ssq[...] / N - m * m
            y = y_s[pl.ds(i * _R, _R), :]
            yn = (y - m) * lax.rsqrt(v + _EPS) * g_ref[...] + be_ref[...]
            z = _leaky(yn) + res_ref[...]

            @pl.when(i == 0)
            def _():
                zsum[...] = jnp.zeros_like(zsum)

            zsum[...] += jnp.sum(z, axis=0, keepdims=True)

            @pl.when(i == nbk - 1)
            def _():
                pooled = zsum[...] / N
                h1 = _leaky(jnp.dot(pooled, w1_ref[...],
                                    preferred_element_type=jnp.float32)
                            + b1_ref[...])
                o = jnp.dot(h1, w2_ref[...],
                            preferred_element_type=jnp.float32) + b2_ref[...]
                nrm = jnp.sqrt(jnp.sum(o * o, axis=1, keepdims=True))
                out_ref[...] = o / jnp.maximum(nrm, 1e-12)

    frozen = lambda p, i: (i * (1 - p) + (nbk - 1) * p, 0)
    return pl.pallas_call(
        body,
        grid=(2, nbk),
        in_specs=[
            pl.BlockSpec((_NC, _R, F), lambda p, i: (0,) + frozen(p, i)),
            pl.BlockSpec((_R, F), frozen),
            pl.BlockSpec((_R, 1), lambda p, i: (i, 0)),
            pl.BlockSpec((1, F), lambda p, i: (0, 0)),
            pl.BlockSpec((1, F), lambda p, i: (0, 0)),
            pl.BlockSpec((1, F), lambda p, i: (0, 0)),
            pl.BlockSpec((_R, F), lambda p, i: (i * p, 0)),
            pl.BlockSpec((F, F), lambda p, i: (0, 0)),
            pl.BlockSpec((1, F), lambda p, i: (0, 0)),
            pl.BlockSpec((F, D), lambda p, i: (0, 0)),
            pl.BlockSpec((1, D), lambda p, i: (0, 0)),
        ],
        out_specs=pl.BlockSpec((1, D), lambda p, i: (0, 0)),
        out_shape=jax.ShapeDtypeStruct((1, D), jnp.float32),
        scratch_shapes=[
            pltpu.VMEM((N, F), jnp.float32),
            pltpu.VMEM((1, F), jnp.float32),
            pltpu.VMEM((1, F), jnp.float32),
            pltpu.VMEM((1, F), jnp.float32),
        ],
    )(sp, hp, dinv, b.reshape(1, F), g.reshape(1, F), be.reshape(1, F), res,
      fcW1, fcb1.reshape(1, F), fcW2, fcb2.reshape(1, D))


def kernel(x, edge_index, W1, b1, W2, b2, W3, b3, g1, be1, g2, be2, g3, be3,
           fcW1, fcb1, fcW2, fcb2):
    N, F = x.shape
    E = edge_index.shape[1]

    ei = edge_index.reshape(2 * E)
    degp = _deg_kernel(N, E)(ei, jnp.zeros((N,), jnp.float32)).reshape(_NC, N)
    xw = _matmul(x, W1)
    hp1, dinv = _scale(xw, degp)

    scat = _scatter_kernel(N, F, E)
    sp1 = scat(hp1, ei)
    z1, hp2 = _layer_tail(sp1, hp1, dinv, b1, g1, be1, None, W2)
    sp2 = scat(hp2, ei)
    z2, hp3 = _layer_tail(sp2, hp2, dinv, b2, g2, be2, z1, W3)
    sp3 = scat(hp3, ei)
    return _head(sp3, hp3, dinv, b3, g3, be3, z2, fcW1, fcb1, fcW2, fcb2)
